# baseline (speedup 1.0000x reference)
import functools

import jax
import jax.numpy as jnp
from jax import lax
from jax.experimental import pallas as pl
from jax.experimental.pallas import tpu as pltpu
from jax.experimental.pallas import tpu_sc as plsc


def _transform_body(ut_ref, vt_ref, wu_ref, wv_ref, tu_ref, tv_ref):
    dn = (((0,), (0,)), ((), ()))
    tu_ref[...] = jnp.maximum(
        lax.dot_general(ut_ref[...], wu_ref[...], dn,
                        preferred_element_type=jnp.float32), 0.0
    ).astype(jnp.bfloat16)
    tv_ref[...] = jnp.maximum(
        lax.dot_general(vt_ref[...], wv_ref[...], dn,
                        preferred_element_type=jnp.float32), 0.0
    ).astype(jnp.bfloat16)


def _transform_tables(u_features, v_features, Wu, Wv, row_block):
    n, d = u_features.shape
    grid = (n + row_block - 1) // row_block
    return pl.pallas_call(
        _transform_body,
        grid=(grid,),
        in_specs=[
            pl.BlockSpec((d, row_block), lambda i: (0, i)),
            pl.BlockSpec((d, row_block), lambda i: (0, i)),
            pl.BlockSpec((d, d), lambda i: (0, 0)),
            pl.BlockSpec((d, d), lambda i: (0, 0)),
        ],
        out_specs=[
            pl.BlockSpec((row_block, d), lambda i: (i, 0)),
            pl.BlockSpec((row_block, d), lambda i: (i, 0)),
        ],
        out_shape=[
            jax.ShapeDtypeStruct((n, d), jnp.bfloat16),
            jax.ShapeDtypeStruct((n, d), jnp.bfloat16),
        ],
    )(u_features.T, v_features.T, Wu, Wv)


def _make_sc_call(B, S, D, NW, Bt, CB):
    NCH = Bt // CB
    G = (CB * S) // 128
    mesh = plsc.VectorSubcoreMesh(core_axis_name="c", subcore_axis_name="s")
    info = plsc.get_sparse_core_info()
    NC = info.num_cores

    def body(tu, tv, uidx, vidx, usup, vsup, uval, vval, wu_t, wi_t,
             self_u, self_v, wsv, sqv, wsu, squ,
             sup_v, val_v, rows_v, wtab_v, ws_st, sq_st, sidx_v, srows_v,
             sems):
        wid = lax.axis_index("s") * NC + lax.axis_index("c")
        lane = jnp.arange(16, dtype=jnp.int32)

        def gather_self(table, idx_hbm, out_hbm):
            pltpu.sync_copy(idx_hbm.at[pl.ds(wid * Bt, Bt)], sidx_v)
            pltpu.async_copy(table.at[sidx_v], srows_v, sems.at[0]).wait()
            pltpu.sync_copy(srows_v, out_hbm.at[pl.ds(wid * Bt, Bt)])

        def do_side(table, sup2, valf, wtab_hbm, ws_out, sq_out):
            pltpu.sync_copy(wtab_hbm, wtab_v)

            def stage(ch, buf):
                row0 = wid * (Bt * S // 128) + ch * G
                pltpu.sync_copy(sup2.at[pl.ds(row0, G)], sup_v.at[buf])
                pltpu.sync_copy(
                    valf.at[pl.ds((wid * Bt + ch * CB) * S, CB * S)],
                    val_v.at[pl.ds(buf * CB * S, CB * S)])
                return [pltpu.async_copy(
                    table.at[sup_v.at[buf, g]],
                    rows_v.at[pl.ds((buf * G + g) * 128, 128)],
                    sems.at[buf]) for g in range(G)]

            def compute(ch, buf):
                def b_body(b, _):
                    def s_body(s, carry):
                        ws0, ws1, ws2, ws3, q0, q1, q2, q3 = carry
                        i = (buf * CB + b) * S + s
                        isplat = jnp.full((16,), 0, jnp.int32) + i
                        vsplat = plsc.load_gather(val_v, [isplat])
                        wbase = vsplat * D
                        a01 = plsc.bitcast(rows_v[i, pl.ds(0, 32)],
                                           jnp.uint32)
                        a23 = plsc.bitcast(rows_v[i, pl.ds(32, 32)],
                                           jnp.uint32)
                        hi = jnp.uint32(0xFFFF0000)
                        r0 = plsc.bitcast(a01 << 16, jnp.float32)
                        r1 = plsc.bitcast(a01 & hi, jnp.float32)
                        r2 = plsc.bitcast(a23 << 16, jnp.float32)
                        r3 = plsc.bitcast(a23 & hi, jnp.float32)
                        w0 = plsc.load_gather(wtab_v, [wbase + lane])
                        w1 = plsc.load_gather(wtab_v, [wbase + lane + 16])
                        w2 = plsc.load_gather(wtab_v, [wbase + lane + 32])
                        w3 = plsc.load_gather(wtab_v, [wbase + lane + 48])
                        return (ws0 + r0 * w0, ws1 + r1 * w1,
                                ws2 + r2 * w2, ws3 + r3 * w3,
                                q0 + r0 * r0, q1 + r1 * r1,
                                q2 + r2 * r2, q3 + r3 * r3)

                    z = jnp.zeros((16,), jnp.float32)
                    acc = lax.fori_loop(0, S, s_body, (z,) * 8)
                    row = ch * CB + b
                    for k in range(4):
                        ws_st[pl.ds(row * D + k * 16, 16)] = acc[k]
                        sq_st[pl.ds(row * D + k * 16, 16)] = acc[4 + k]
                    return 0

                lax.fori_loop(0, CB, b_body, 0)

            pending = stage(0, 0)
            for ch in range(NCH):
                nxt = None
                if ch + 1 < NCH:
                    nxt = stage(ch + 1, (ch + 1) % 2)
                for cp in pending:
                    cp.wait()
                compute(ch, ch % 2)
                pending = nxt
            pltpu.sync_copy(ws_st, ws_out.at[pl.ds(wid * Bt * D, Bt * D)])
            pltpu.sync_copy(sq_st, sq_out.at[pl.ds(wid * Bt * D, Bt * D)])

        gather_self(tu, uidx, self_u)
        gather_self(tv, vidx, self_v)
        do_side(tv, vsup, vval, wi_t, wsv, sqv)
        do_side(tu, usup, uval, wu_t, wsu, squ)

    return pl.kernel(
        body,
        out_type=[
            jax.ShapeDtypeStruct((B, D), jnp.bfloat16),
            jax.ShapeDtypeStruct((B, D), jnp.bfloat16),
            jax.ShapeDtypeStruct((B * D,), jnp.float32),
            jax.ShapeDtypeStruct((B * D,), jnp.float32),
            jax.ShapeDtypeStruct((B * D,), jnp.float32),
            jax.ShapeDtypeStruct((B * D,), jnp.float32),
        ],
        mesh=mesh,
        compiler_params=pltpu.CompilerParams(
            use_tc_tiling_on_sc=False, needs_layout_passes=False),
        scratch_types=[
            pltpu.VMEM((2, G, 128), jnp.int32),
            pltpu.VMEM((2 * CB * S,), jnp.int32),
            pltpu.VMEM((2 * CB * S, D), jnp.bfloat16),
            pltpu.VMEM((5 * D,), jnp.float32),
            pltpu.VMEM((Bt * D,), jnp.float32),
            pltpu.VMEM((Bt * D,), jnp.float32),
            pltpu.VMEM((Bt,), jnp.int32),
            pltpu.VMEM((Bt, D), jnp.bfloat16),
            pltpu.SemaphoreType.DMA((2,)),
        ],
    )


def _l2rows(x):
    sq = jnp.sum(x * x, axis=1, keepdims=True)
    return x * lax.rsqrt(jnp.maximum(sq, 1e-12))


def _finish_body(inv_s_ref, su_ref, sv_ref, wsv_ref, sqv_ref, wsu_ref, squ_ref,
                 wvagg_ref, wuagg_ref, wout_ref, out_ref):
    inv_s = inv_s_ref[0]
    u0 = _l2rows(su_ref[...].astype(jnp.float32))
    i0 = _l2rows(sv_ref[...].astype(jnp.float32))
    nv = wsv_ref[...] * lax.rsqrt(jnp.maximum(sqv_ref[...], 1e-12)) * inv_s
    nu = wsu_ref[...] * lax.rsqrt(jnp.maximum(squ_ref[...], 1e-12)) * inv_s
    hu = jnp.concatenate([u0, nv], axis=1)
    hi = jnp.concatenate([i0, nu], axis=1)
    uvec = _l2rows(jnp.maximum(
        jnp.dot(hu, wvagg_ref[...], preferred_element_type=jnp.float32), 0.0))
    ivec = _l2rows(jnp.maximum(
        jnp.dot(hi, wuagg_ref[...], preferred_element_type=jnp.float32), 0.0))
    out_ref[...] = jnp.dot(jnp.concatenate([uvec, ivec], axis=1),
                           wout_ref[...], preferred_element_type=jnp.float32)


def _finish(S, self_u, self_v, wsv, sqv, wsu, squ, Wv_agg, Wu_agg, Wout):
    B, D = self_u.shape
    inv_s = jnp.full((1,), 1.0 / S, jnp.float32)
    return pl.pallas_call(
        _finish_body,
        in_specs=[pl.BlockSpec(memory_space=pltpu.SMEM)] + [
            pl.BlockSpec(x.shape, lambda: (0,) * x.ndim)
            for x in (self_u, self_v, wsv, sqv, wsu, squ, Wv_agg, Wu_agg, Wout)],
        out_specs=pl.BlockSpec((B, Wout.shape[1]), lambda: (0, 0)),
        out_shape=jax.ShapeDtypeStruct((B, Wout.shape[1]), jnp.float32),
    )(inv_s, self_u, self_v, wsv, sqv, wsu, squ, Wv_agg, Wu_agg, Wout)


def kernel(u_features, v_features, Wu, Wv, Wout, i_edge_weights, u_edge_weights,
           Wv_agg, Wu_agg, u_indices, v_indices, u_supports, v_supports,
           user_support_val, item_support_val):
    B, S = u_supports.shape
    D = Wu.shape[0]
    NW = 32
    Bt = B // NW
    CB = 16

    Tu, Tv = _transform_tables(u_features, v_features, Wu, Wv, row_block=8192)

    half = D // 2
    perm = jnp.concatenate([
        jnp.arange(0, half, 2), jnp.arange(1, half, 2),
        jnp.arange(half, D, 2), jnp.arange(half + 1, D, 2)])

    sc_call = _make_sc_call(B, S, D, NW, Bt, CB)
    i32 = jnp.int32
    self_u, self_v, wsv, sqv, wsu, squ = sc_call(
        Tu, Tv,
        u_indices.astype(i32), v_indices.astype(i32),
        u_supports.astype(i32).reshape(-1, 128),
        v_supports.astype(i32).reshape(-1, 128),
        user_support_val.astype(i32).reshape(-1),
        item_support_val.astype(i32).reshape(-1),
        u_edge_weights[:, perm].reshape(-1),
        i_edge_weights[:, perm].reshape(-1),
    )

    wvagg_p = jnp.concatenate([Wv_agg[:D], Wv_agg[D:][perm]])
    wuagg_p = jnp.concatenate([Wu_agg[:D], Wu_agg[D:][perm]])
    return _finish(S, self_u, self_v,
                   wsv.reshape(B, D), sqv.reshape(B, D),
                   wsu.reshape(B, D), squ.reshape(B, D),
                   wvagg_p, wuagg_p, Wout)


# speedup vs baseline: 1.1232x; 1.1232x over previous
import functools

import jax
import jax.numpy as jnp
from jax import lax
from jax.experimental import pallas as pl
from jax.experimental.pallas import tpu as pltpu
from jax.experimental.pallas import tpu_sc as plsc


def _transform_body(ut_ref, vt_ref, wue_ref, wuo_ref, wve_ref, wvo_ref,
                    tu_ref, tv_ref):
    dn = (((0,), (0,)), ((), ()))
    hi = jnp.int32(-65536)

    def pack(x, we, wo):
        e = jnp.maximum(lax.dot_general(
            x, we, dn, preferred_element_type=jnp.float32), 0.0)
        o = jnp.maximum(lax.dot_general(
            x, wo, dn, preferred_element_type=jnp.float32), 0.0)
        eb = lax.bitcast_convert_type(
            e.astype(jnp.bfloat16).astype(jnp.float32), jnp.int32)
        ob = lax.bitcast_convert_type(
            o.astype(jnp.bfloat16).astype(jnp.float32), jnp.int32)
        return lax.shift_right_logical(eb, 16) | (ob & hi)

    tu_ref[...] = pack(ut_ref[...], wue_ref[...], wuo_ref[...])
    tv_ref[...] = pack(vt_ref[...], wve_ref[...], wvo_ref[...])


def _transform_tables(u_features, v_features, Wu, Wv, row_block):
    n, d = u_features.shape
    h = d // 2
    grid = (n + row_block - 1) // row_block
    return pl.pallas_call(
        _transform_body,
        grid=(grid,),
        in_specs=[
            pl.BlockSpec((d, row_block), lambda i: (0, i)),
            pl.BlockSpec((d, row_block), lambda i: (0, i)),
            pl.BlockSpec((d, h), lambda i: (0, 0)),
            pl.BlockSpec((d, h), lambda i: (0, 0)),
            pl.BlockSpec((d, h), lambda i: (0, 0)),
            pl.BlockSpec((d, h), lambda i: (0, 0)),
        ],
        out_specs=[
            pl.BlockSpec((row_block, h), lambda i: (i, 0)),
            pl.BlockSpec((row_block, h), lambda i: (i, 0)),
        ],
        out_shape=[
            jax.ShapeDtypeStruct((n, h), jnp.int32),
            jax.ShapeDtypeStruct((n, h), jnp.int32),
        ],
    )(u_features.T, v_features.T,
      Wu[:, 0::2], Wu[:, 1::2], Wv[:, 0::2], Wv[:, 1::2])


def _make_sc_call(B, S, D, NW, Bt, CB):
    NCH = Bt // CB
    G = (CB * S) // 128
    mesh = plsc.VectorSubcoreMesh(core_axis_name="c", subcore_axis_name="s")
    info = plsc.get_sparse_core_info()
    NC = info.num_cores

    def body(tu, tv, uidx, vidx, usup, vsup, uval, vval, wu_t, wi_t,
             self_u, self_v, wsv, sqv, wsu, squ,
             sup_v, val_v, rows_v, wtab_v, ws_st, sq_st, sidx_v, srows_v,
             sems):
        wid = lax.axis_index("s") * NC + lax.axis_index("c")
        lane = jnp.arange(16, dtype=jnp.int32)

        def gather_self(table, idx_hbm, out_hbm):
            pltpu.sync_copy(idx_hbm.at[pl.ds(wid * Bt, Bt)], sidx_v)
            pltpu.async_copy(table.at[sidx_v], srows_v, sems.at[0]).wait()
            pltpu.sync_copy(srows_v, out_hbm.at[pl.ds(wid * Bt, Bt)])

        def do_side(table, sup2, valf, wtab_hbm, ws_out, sq_out):
            pltpu.sync_copy(wtab_hbm, wtab_v)

            def stage(ch, buf):
                row0 = wid * (Bt * S // 128) + ch * G
                pltpu.sync_copy(sup2.at[pl.ds(row0, G)], sup_v.at[buf])
                pltpu.sync_copy(
                    valf.at[pl.ds((wid * Bt + ch * CB) * S, CB * S)],
                    val_v.at[pl.ds(buf * CB * S, CB * S)])
                return [pltpu.async_copy(
                    table.at[sup_v.at[buf, g]],
                    rows_v.at[pl.ds((buf * G + g) * 128, 128)],
                    sems.at[buf]) for g in range(G)]

            def compute(ch, buf):
                def b_body(b, _):
                    def s_body(s, carry):
                        ws0, ws1, ws2, ws3, q0, q1, q2, q3 = carry
                        i = (buf * CB + b) * S + s
                        isplat = jnp.full((16,), 0, jnp.int32) + i
                        vsplat = plsc.load_gather(val_v, [isplat])
                        wbase = vsplat * D
                        a01 = rows_v[i, pl.ds(0, 16)]
                        a23 = rows_v[i, pl.ds(16, 16)]
                        hi = jnp.int32(-65536)
                        r0 = plsc.bitcast(a01 << 16, jnp.float32)
                        r1 = plsc.bitcast(a01 & hi, jnp.float32)
                        r2 = plsc.bitcast(a23 << 16, jnp.float32)
                        r3 = plsc.bitcast(a23 & hi, jnp.float32)
                        w0 = plsc.load_gather(wtab_v, [wbase + lane])
                        w1 = plsc.load_gather(wtab_v, [wbase + lane + 16])
                        w2 = plsc.load_gather(wtab_v, [wbase + lane + 32])
                        w3 = plsc.load_gather(wtab_v, [wbase + lane + 48])
                        return (ws0 + r0 * w0, ws1 + r1 * w1,
                                ws2 + r2 * w2, ws3 + r3 * w3,
                                q0 + r0 * r0, q1 + r1 * r1,
                                q2 + r2 * r2, q3 + r3 * r3)

                    z = jnp.zeros((16,), jnp.float32)
                    acc = lax.fori_loop(0, S, s_body, (z,) * 8)
                    row = ch * CB + b
                    for k in range(4):
                        ws_st[pl.ds(row * D + k * 16, 16)] = acc[k]
                        sq_st[pl.ds(row * D + k * 16, 16)] = acc[4 + k]
                    return 0

                lax.fori_loop(0, CB, b_body, 0)

            pending = stage(0, 0)
            for ch in range(NCH):
                nxt = None
                if ch + 1 < NCH:
                    nxt = stage(ch + 1, (ch + 1) % 2)
                for cp in pending:
                    cp.wait()
                compute(ch, ch % 2)
                pending = nxt
            pltpu.sync_copy(ws_st, ws_out.at[pl.ds(wid * Bt * D, Bt * D)])
            pltpu.sync_copy(sq_st, sq_out.at[pl.ds(wid * Bt * D, Bt * D)])

        gather_self(tu, uidx, self_u)
        gather_self(tv, vidx, self_v)
        do_side(tv, vsup, vval, wi_t, wsv, sqv)
        do_side(tu, usup, uval, wu_t, wsu, squ)

    return pl.kernel(
        body,
        out_type=[
            jax.ShapeDtypeStruct((B, D // 2), jnp.int32),
            jax.ShapeDtypeStruct((B, D // 2), jnp.int32),
            jax.ShapeDtypeStruct((B * D,), jnp.float32),
            jax.ShapeDtypeStruct((B * D,), jnp.float32),
            jax.ShapeDtypeStruct((B * D,), jnp.float32),
            jax.ShapeDtypeStruct((B * D,), jnp.float32),
        ],
        mesh=mesh,
        compiler_params=pltpu.CompilerParams(
            use_tc_tiling_on_sc=False, needs_layout_passes=False),
        scratch_types=[
            pltpu.VMEM((2, G, 128), jnp.int32),
            pltpu.VMEM((2 * CB * S,), jnp.int32),
            pltpu.VMEM((2 * CB * S, D // 2), jnp.int32),
            pltpu.VMEM((5 * D,), jnp.float32),
            pltpu.VMEM((Bt * D,), jnp.float32),
            pltpu.VMEM((Bt * D,), jnp.float32),
            pltpu.VMEM((Bt,), jnp.int32),
            pltpu.VMEM((Bt, D // 2), jnp.int32),
            pltpu.SemaphoreType.DMA((2,)),
        ],
    )


def _l2rows(x):
    sq = jnp.sum(x * x, axis=1, keepdims=True)
    return x * lax.rsqrt(jnp.maximum(sq, 1e-12))


def _unpack_packed(a):
    hi = jnp.int32(-65536)
    e = lax.bitcast_convert_type(lax.shift_left(a, 16), jnp.float32)
    o = lax.bitcast_convert_type(a & hi, jnp.float32)
    return jnp.concatenate([e, o], axis=1)


def _finish_body(inv_s_ref, su_ref, sv_ref, wsv_ref, sqv_ref, wsu_ref, squ_ref,
                 wvagg_ref, wuagg_ref, wout_ref, out_ref):
    inv_s = inv_s_ref[0]
    u0 = _l2rows(_unpack_packed(su_ref[...]))
    i0 = _l2rows(_unpack_packed(sv_ref[...]))
    nv = wsv_ref[...] * lax.rsqrt(jnp.maximum(sqv_ref[...], 1e-12)) * inv_s
    nu = wsu_ref[...] * lax.rsqrt(jnp.maximum(squ_ref[...], 1e-12)) * inv_s
    hu = jnp.concatenate([u0, nv], axis=1)
    hi = jnp.concatenate([i0, nu], axis=1)
    uvec = _l2rows(jnp.maximum(
        jnp.dot(hu, wvagg_ref[...], preferred_element_type=jnp.float32), 0.0))
    ivec = _l2rows(jnp.maximum(
        jnp.dot(hi, wuagg_ref[...], preferred_element_type=jnp.float32), 0.0))
    out_ref[...] = jnp.dot(jnp.concatenate([uvec, ivec], axis=1),
                           wout_ref[...], preferred_element_type=jnp.float32)


def _finish(S, self_u, self_v, wsv, sqv, wsu, squ, Wv_agg, Wu_agg, Wout):
    B, D = self_u.shape
    inv_s = jnp.full((1,), 1.0 / S, jnp.float32)
    return pl.pallas_call(
        _finish_body,
        in_specs=[pl.BlockSpec(memory_space=pltpu.SMEM)] + [
            pl.BlockSpec(x.shape, lambda: (0,) * x.ndim)
            for x in (self_u, self_v, wsv, sqv, wsu, squ, Wv_agg, Wu_agg, Wout)],
        out_specs=pl.BlockSpec((B, Wout.shape[1]), lambda: (0, 0)),
        out_shape=jax.ShapeDtypeStruct((B, Wout.shape[1]), jnp.float32),
    )(inv_s, self_u, self_v, wsv, sqv, wsu, squ, Wv_agg, Wu_agg, Wout)


def kernel(u_features, v_features, Wu, Wv, Wout, i_edge_weights, u_edge_weights,
           Wv_agg, Wu_agg, u_indices, v_indices, u_supports, v_supports,
           user_support_val, item_support_val):
    B, S = u_supports.shape
    D = Wu.shape[0]
    NW = 32
    Bt = B // NW
    CB = 16

    Tu, Tv = _transform_tables(u_features, v_features, Wu, Wv, row_block=8192)

    half = D // 2
    perm = jnp.concatenate([
        jnp.arange(0, half, 2), jnp.arange(1, half, 2),
        jnp.arange(half, D, 2), jnp.arange(half + 1, D, 2)])

    sc_call = _make_sc_call(B, S, D, NW, Bt, CB)
    i32 = jnp.int32
    self_u, self_v, wsv, sqv, wsu, squ = sc_call(
        Tu, Tv,
        u_indices.astype(i32), v_indices.astype(i32),
        u_supports.astype(i32).reshape(-1, 128),
        v_supports.astype(i32).reshape(-1, 128),
        user_support_val.astype(i32).reshape(-1),
        item_support_val.astype(i32).reshape(-1),
        u_edge_weights[:, perm].reshape(-1),
        i_edge_weights[:, perm].reshape(-1),
    )

    perm2 = jnp.concatenate([jnp.arange(0, D, 2), jnp.arange(1, D, 2)])
    wvagg_p = jnp.concatenate([Wv_agg[:D][perm2], Wv_agg[D:][perm]])
    wuagg_p = jnp.concatenate([Wu_agg[:D][perm2], Wu_agg[D:][perm]])
    return _finish(S, self_u, self_v,
                   wsv.reshape(B, D), sqv.reshape(B, D),
                   wsu.reshape(B, D), squ.reshape(B, D),
                   wvagg_p, wuagg_p, Wout)


# speedup vs baseline: 1.1257x; 1.0022x over previous
import functools

import jax
import jax.numpy as jnp
from jax import lax
from jax.experimental import pallas as pl
from jax.experimental.pallas import tpu as pltpu
from jax.experimental.pallas import tpu_sc as plsc


def _transform_body(ut_ref, vt_ref, wue_ref, wuo_ref, wve_ref, wvo_ref,
                    tu_ref, tv_ref):
    dn = (((0,), (0,)), ((), ()))
    hi = jnp.int32(-65536)

    def pack(x, we, wo):
        e = jnp.maximum(lax.dot_general(
            x, we, dn, preferred_element_type=jnp.float32), 0.0)
        o = jnp.maximum(lax.dot_general(
            x, wo, dn, preferred_element_type=jnp.float32), 0.0)
        eb = lax.bitcast_convert_type(
            e.astype(jnp.bfloat16).astype(jnp.float32), jnp.int32)
        ob = lax.bitcast_convert_type(
            o.astype(jnp.bfloat16).astype(jnp.float32), jnp.int32)
        return lax.shift_right_logical(eb, 16) | (ob & hi)

    tu_ref[...] = pack(ut_ref[...], wue_ref[...], wuo_ref[...])
    tv_ref[...] = pack(vt_ref[...], wve_ref[...], wvo_ref[...])


def _transform_tables(u_features, v_features, Wu, Wv, row_block):
    n, d = u_features.shape
    h = d // 2
    grid = (n + row_block - 1) // row_block
    return pl.pallas_call(
        _transform_body,
        grid=(grid,),
        in_specs=[
            pl.BlockSpec((d, row_block), lambda i: (0, i)),
            pl.BlockSpec((d, row_block), lambda i: (0, i)),
            pl.BlockSpec((d, h), lambda i: (0, 0)),
            pl.BlockSpec((d, h), lambda i: (0, 0)),
            pl.BlockSpec((d, h), lambda i: (0, 0)),
            pl.BlockSpec((d, h), lambda i: (0, 0)),
        ],
        out_specs=[
            pl.BlockSpec((row_block, h), lambda i: (i, 0)),
            pl.BlockSpec((row_block, h), lambda i: (i, 0)),
        ],
        out_shape=[
            jax.ShapeDtypeStruct((n, h), jnp.int32),
            jax.ShapeDtypeStruct((n, h), jnp.int32),
        ],
    )(u_features.T, v_features.T,
      Wu[:, 0::2], Wu[:, 1::2], Wv[:, 0::2], Wv[:, 1::2])


def _make_sc_call(B, S, D, NW, Bt, CB):
    NCH = Bt // CB
    G = (CB * S) // 128
    mesh = plsc.VectorSubcoreMesh(core_axis_name="c", subcore_axis_name="s")
    info = plsc.get_sparse_core_info()
    NC = info.num_cores

    def body(tu, tv, uidx, vidx, usup, vsup, uval, vval, wu_t, wi_t,
             self_u, self_v, wsv, sqv, wsu, squ,
             sup_v, val_v, rows_v, wtab_v, ws_st, sq_st, sidx_v, srows_v,
             sems):
        wid = lax.axis_index("s") * NC + lax.axis_index("c")
        lane = jnp.arange(16, dtype=jnp.int32)

        def gather_self(table, idx_hbm, out_hbm):
            pltpu.sync_copy(idx_hbm.at[pl.ds(wid * Bt, Bt)], sidx_v)
            pltpu.async_copy(table.at[sidx_v], srows_v, sems.at[0]).wait()
            pltpu.sync_copy(srows_v, out_hbm.at[pl.ds(wid * Bt, Bt)])

        def do_side(table, sup2, valf, wtab_hbm, ws_out, sq_out):
            pltpu.sync_copy(wtab_hbm, wtab_v)

            def stage(ch, buf):
                row0 = wid * (Bt * S // 128) + ch * G
                pltpu.sync_copy(sup2.at[pl.ds(row0, G)], sup_v.at[buf])
                pltpu.sync_copy(
                    valf.at[pl.ds((wid * Bt + ch * CB) * S, CB * S)],
                    val_v.at[pl.ds(buf * CB * S, CB * S)])
                return [pltpu.async_copy(
                    table.at[sup_v.at[buf, g]],
                    rows_v.at[pl.ds((buf * G + g) * 128, 128)],
                    sems.at[buf]) for g in range(G)]

            def compute(ch, buf):
                hi = jnp.int32(-65536)
                dn = lax.GatherDimensionNumbers(
                    offset_dims=(), collapsed_slice_dims=(0,),
                    start_index_map=(0,))

                def b_body(b, _):
                    def sg_body(sg, carry):
                        base = (buf * CB + b) * S + sg * 16
                        wb = val_v[pl.ds(base, 16)] * D
                        accs = list(carry)
                        for j in range(16):
                            i = base + j
                            wbase = lax.gather(
                                wb, jnp.full((16, 1), j, jnp.int32), dn, (1,),
                                mode=lax.GatherScatterMode.PROMISE_IN_BOUNDS)
                            a01 = rows_v[i, pl.ds(0, 16)]
                            a23 = rows_v[i, pl.ds(16, 16)]
                            r0 = plsc.bitcast(a01 << 16, jnp.float32)
                            r1 = plsc.bitcast(a01 & hi, jnp.float32)
                            r2 = plsc.bitcast(a23 << 16, jnp.float32)
                            r3 = plsc.bitcast(a23 & hi, jnp.float32)
                            w0 = plsc.load_gather(wtab_v, [wbase + lane])
                            w1 = plsc.load_gather(wtab_v, [wbase + lane + 16])
                            w2 = plsc.load_gather(wtab_v, [wbase + lane + 32])
                            w3 = plsc.load_gather(wtab_v, [wbase + lane + 48])
                            accs = [accs[0] + r0 * w0, accs[1] + r1 * w1,
                                    accs[2] + r2 * w2, accs[3] + r3 * w3,
                                    accs[4] + r0 * r0, accs[5] + r1 * r1,
                                    accs[6] + r2 * r2, accs[7] + r3 * r3]
                        return tuple(accs)

                    z = jnp.zeros((16,), jnp.float32)
                    acc = lax.fori_loop(0, S // 16, sg_body, (z,) * 8)
                    row = ch * CB + b
                    for k in range(4):
                        ws_st[pl.ds(row * D + k * 16, 16)] = acc[k]
                        sq_st[pl.ds(row * D + k * 16, 16)] = acc[4 + k]
                    return 0

                lax.fori_loop(0, CB, b_body, 0)

            pending = stage(0, 0)
            for ch in range(NCH):
                nxt = None
                if ch + 1 < NCH:
                    nxt = stage(ch + 1, (ch + 1) % 2)
                for cp in pending:
                    cp.wait()
                compute(ch, ch % 2)
                pending = nxt
            pltpu.sync_copy(ws_st, ws_out.at[pl.ds(wid * Bt * D, Bt * D)])
            pltpu.sync_copy(sq_st, sq_out.at[pl.ds(wid * Bt * D, Bt * D)])

        gather_self(tu, uidx, self_u)
        gather_self(tv, vidx, self_v)
        do_side(tv, vsup, vval, wi_t, wsv, sqv)
        do_side(tu, usup, uval, wu_t, wsu, squ)

    return pl.kernel(
        body,
        out_type=[
            jax.ShapeDtypeStruct((B, D // 2), jnp.int32),
            jax.ShapeDtypeStruct((B, D // 2), jnp.int32),
            jax.ShapeDtypeStruct((B * D,), jnp.float32),
            jax.ShapeDtypeStruct((B * D,), jnp.float32),
            jax.ShapeDtypeStruct((B * D,), jnp.float32),
            jax.ShapeDtypeStruct((B * D,), jnp.float32),
        ],
        mesh=mesh,
        compiler_params=pltpu.CompilerParams(
            use_tc_tiling_on_sc=False, needs_layout_passes=False),
        scratch_types=[
            pltpu.VMEM((2, G, 128), jnp.int32),
            pltpu.VMEM((2 * CB * S,), jnp.int32),
            pltpu.VMEM((2 * CB * S, D // 2), jnp.int32),
            pltpu.VMEM((5 * D,), jnp.float32),
            pltpu.VMEM((Bt * D,), jnp.float32),
            pltpu.VMEM((Bt * D,), jnp.float32),
            pltpu.VMEM((Bt,), jnp.int32),
            pltpu.VMEM((Bt, D // 2), jnp.int32),
            pltpu.SemaphoreType.DMA((2,)),
        ],
    )


def _l2rows(x):
    sq = jnp.sum(x * x, axis=1, keepdims=True)
    return x * lax.rsqrt(jnp.maximum(sq, 1e-12))


def _unpack_packed(a):
    hi = jnp.int32(-65536)
    e = lax.bitcast_convert_type(lax.shift_left(a, 16), jnp.float32)
    o = lax.bitcast_convert_type(a & hi, jnp.float32)
    return jnp.concatenate([e, o], axis=1)


def _finish_body(inv_s_ref, su_ref, sv_ref, wsv_ref, sqv_ref, wsu_ref, squ_ref,
                 wvagg_ref, wuagg_ref, wout_ref, out_ref):
    inv_s = inv_s_ref[0]
    u0 = _l2rows(_unpack_packed(su_ref[...]))
    i0 = _l2rows(_unpack_packed(sv_ref[...]))
    nv = wsv_ref[...] * lax.rsqrt(jnp.maximum(sqv_ref[...], 1e-12)) * inv_s
    nu = wsu_ref[...] * lax.rsqrt(jnp.maximum(squ_ref[...], 1e-12)) * inv_s
    hu = jnp.concatenate([u0, nv], axis=1)
    hi = jnp.concatenate([i0, nu], axis=1)
    uvec = _l2rows(jnp.maximum(
        jnp.dot(hu, wvagg_ref[...], preferred_element_type=jnp.float32), 0.0))
    ivec = _l2rows(jnp.maximum(
        jnp.dot(hi, wuagg_ref[...], preferred_element_type=jnp.float32), 0.0))
    out_ref[...] = jnp.dot(jnp.concatenate([uvec, ivec], axis=1),
                           wout_ref[...], preferred_element_type=jnp.float32)


def _finish(S, self_u, self_v, wsv, sqv, wsu, squ, Wv_agg, Wu_agg, Wout):
    B, D = self_u.shape
    inv_s = jnp.full((1,), 1.0 / S, jnp.float32)
    return pl.pallas_call(
        _finish_body,
        in_specs=[pl.BlockSpec(memory_space=pltpu.SMEM)] + [
            pl.BlockSpec(x.shape, lambda: (0,) * x.ndim)
            for x in (self_u, self_v, wsv, sqv, wsu, squ, Wv_agg, Wu_agg, Wout)],
        out_specs=pl.BlockSpec((B, Wout.shape[1]), lambda: (0, 0)),
        out_shape=jax.ShapeDtypeStruct((B, Wout.shape[1]), jnp.float32),
    )(inv_s, self_u, self_v, wsv, sqv, wsu, squ, Wv_agg, Wu_agg, Wout)


def kernel(u_features, v_features, Wu, Wv, Wout, i_edge_weights, u_edge_weights,
           Wv_agg, Wu_agg, u_indices, v_indices, u_supports, v_supports,
           user_support_val, item_support_val):
    B, S = u_supports.shape
    D = Wu.shape[0]
    NW = 32
    Bt = B // NW
    CB = 16

    Tu, Tv = _transform_tables(u_features, v_features, Wu, Wv, row_block=8192)

    half = D // 2
    perm = jnp.concatenate([
        jnp.arange(0, half, 2), jnp.arange(1, half, 2),
        jnp.arange(half, D, 2), jnp.arange(half + 1, D, 2)])

    sc_call = _make_sc_call(B, S, D, NW, Bt, CB)
    i32 = jnp.int32
    self_u, self_v, wsv, sqv, wsu, squ = sc_call(
        Tu, Tv,
        u_indices.astype(i32), v_indices.astype(i32),
        u_supports.astype(i32).reshape(-1, 128),
        v_supports.astype(i32).reshape(-1, 128),
        user_support_val.astype(i32).reshape(-1),
        item_support_val.astype(i32).reshape(-1),
        u_edge_weights[:, perm].reshape(-1),
        i_edge_weights[:, perm].reshape(-1),
    )

    perm2 = jnp.concatenate([jnp.arange(0, D, 2), jnp.arange(1, D, 2)])
    wvagg_p = jnp.concatenate([Wv_agg[:D][perm2], Wv_agg[D:][perm]])
    wuagg_p = jnp.concatenate([Wu_agg[:D][perm2], Wu_agg[D:][perm]])
    return _finish(S, self_u, self_v,
                   wsv.reshape(B, D), sqv.reshape(B, D),
                   wsu.reshape(B, D), squ.reshape(B, D),
                   wvagg_p, wuagg_p, Wout)


# speedup vs baseline: 1.2620x; 1.1211x over previous
import functools

import jax
import jax.numpy as jnp
from jax import lax
from jax.experimental import pallas as pl
from jax.experimental.pallas import tpu as pltpu
from jax.experimental.pallas import tpu_sc as plsc


def _transform_body(xt_ref, we_ref, wo_ref, t_ref):
    dn = (((0,), (0,)), ((), ()))
    hi = jnp.int32(-65536)
    x = xt_ref[...]
    e = jnp.maximum(lax.dot_general(
        x, we_ref[...], dn, preferred_element_type=jnp.float32), 0.0)
    o = jnp.maximum(lax.dot_general(
        x, wo_ref[...], dn, preferred_element_type=jnp.float32), 0.0)
    eb = lax.bitcast_convert_type(
        e.astype(jnp.bfloat16).astype(jnp.float32), jnp.int32)
    ob = lax.bitcast_convert_type(
        o.astype(jnp.bfloat16).astype(jnp.float32), jnp.int32)
    t_ref[...] = lax.shift_right_logical(eb, 16) | (ob & hi)


def _transform_table(features, W, row_block):
    n, d = features.shape
    h = d // 2
    grid = (n + row_block - 1) // row_block
    return pl.pallas_call(
        _transform_body,
        grid=(grid,),
        in_specs=[
            pl.BlockSpec((d, row_block), lambda i: (0, i)),
            pl.BlockSpec((d, h), lambda i: (0, 0)),
            pl.BlockSpec((d, h), lambda i: (0, 0)),
        ],
        out_specs=pl.BlockSpec((row_block, h), lambda i: (i, 0)),
        out_shape=jax.ShapeDtypeStruct((n, h), jnp.int32),
    )(features.T, W[:, 0::2], W[:, 1::2])


def _make_sc_call(B, S, D, NW, Bt, CB):
    NCH = Bt // CB
    G = (CB * S) // 128
    mesh = plsc.VectorSubcoreMesh(core_axis_name="c", subcore_axis_name="s")
    info = plsc.get_sparse_core_info()
    NC = info.num_cores

    def body(table, sidx, sup2, valf, wtab_hbm,
             self_out, ws_o, sq_o,
             sup_v, val_v, rows_v, wtab_v, ws_st, sq_st, sidx_v, srows_v,
             sems):
        wid = lax.axis_index("s") * NC + lax.axis_index("c")
        lane = jnp.arange(16, dtype=jnp.int32)

        def gather_self(table, idx_hbm, out_hbm):
            pltpu.sync_copy(idx_hbm.at[pl.ds(wid * Bt, Bt)], sidx_v)
            pltpu.async_copy(table.at[sidx_v], srows_v, sems.at[0]).wait()
            pltpu.sync_copy(srows_v, out_hbm.at[pl.ds(wid * Bt, Bt)])

        def do_side(table, sup2, valf, wtab_hbm, ws_out, sq_out):
            pltpu.sync_copy(wtab_hbm, wtab_v)

            def stage(ch, buf):
                row0 = wid * (Bt * S // 128) + ch * G
                pltpu.sync_copy(sup2.at[pl.ds(row0, G)], sup_v.at[buf])
                pltpu.sync_copy(
                    valf.at[pl.ds((wid * Bt + ch * CB) * S, CB * S)],
                    val_v.at[pl.ds(buf * CB * S, CB * S)])
                return [pltpu.async_copy(
                    table.at[sup_v.at[buf, g]],
                    rows_v.at[pl.ds((buf * G + g) * 128, 128)],
                    sems.at[buf]) for g in range(G)]

            def compute(ch, buf):
                hi = jnp.int32(-65536)
                dn = lax.GatherDimensionNumbers(
                    offset_dims=(), collapsed_slice_dims=(0,),
                    start_index_map=(0,))

                def b_body(b, _):
                    def sg_body(sg, carry):
                        base = (buf * CB + b) * S + sg * 16
                        wb = val_v[pl.ds(base, 16)] * D
                        accs = list(carry)
                        for j in range(16):
                            i = base + j
                            wbase = lax.gather(
                                wb, jnp.full((16, 1), j, jnp.int32), dn, (1,),
                                mode=lax.GatherScatterMode.PROMISE_IN_BOUNDS)
                            a01 = rows_v[i, pl.ds(0, 16)]
                            a23 = rows_v[i, pl.ds(16, 16)]
                            r0 = plsc.bitcast(a01 << 16, jnp.float32)
                            r1 = plsc.bitcast(a01 & hi, jnp.float32)
                            r2 = plsc.bitcast(a23 << 16, jnp.float32)
                            r3 = plsc.bitcast(a23 & hi, jnp.float32)
                            w0 = plsc.load_gather(wtab_v, [wbase + lane])
                            w1 = plsc.load_gather(wtab_v, [wbase + lane + 16])
                            w2 = plsc.load_gather(wtab_v, [wbase + lane + 32])
                            w3 = plsc.load_gather(wtab_v, [wbase + lane + 48])
                            accs = [accs[0] + r0 * w0, accs[1] + r1 * w1,
                                    accs[2] + r2 * w2, accs[3] + r3 * w3,
                                    accs[4] + r0 * r0, accs[5] + r1 * r1,
                                    accs[6] + r2 * r2, accs[7] + r3 * r3]
                        return tuple(accs)

                    z = jnp.zeros((16,), jnp.float32)
                    acc = lax.fori_loop(0, S // 16, sg_body, (z,) * 8)
                    row = ch * CB + b
                    for k in range(4):
                        ws_st[pl.ds(row * D + k * 16, 16)] = acc[k]
                        sq_st[pl.ds(row * D + k * 16, 16)] = acc[4 + k]
                    return 0

                lax.fori_loop(0, CB, b_body, 0)

            pending = stage(0, 0)
            for ch in range(NCH):
                nxt = None
                if ch + 1 < NCH:
                    nxt = stage(ch + 1, (ch + 1) % 2)
                for cp in pending:
                    cp.wait()
                compute(ch, ch % 2)
                pending = nxt
            pltpu.sync_copy(ws_st, ws_out.at[pl.ds(wid * Bt * D, Bt * D)])
            pltpu.sync_copy(sq_st, sq_out.at[pl.ds(wid * Bt * D, Bt * D)])

        gather_self(table, sidx, self_out)
        do_side(table, sup2, valf, wtab_hbm, ws_o, sq_o)

    return pl.kernel(
        body,
        out_type=[
            jax.ShapeDtypeStruct((B, D // 2), jnp.int32),
            jax.ShapeDtypeStruct((B * D,), jnp.float32),
            jax.ShapeDtypeStruct((B * D,), jnp.float32),
        ],
        mesh=mesh,
        compiler_params=pltpu.CompilerParams(
            use_tc_tiling_on_sc=False, needs_layout_passes=False),
        scratch_types=[
            pltpu.VMEM((2, G, 128), jnp.int32),
            pltpu.VMEM((2 * CB * S,), jnp.int32),
            pltpu.VMEM((2 * CB * S, D // 2), jnp.int32),
            pltpu.VMEM((5 * D,), jnp.float32),
            pltpu.VMEM((Bt * D,), jnp.float32),
            pltpu.VMEM((Bt * D,), jnp.float32),
            pltpu.VMEM((Bt,), jnp.int32),
            pltpu.VMEM((Bt, D // 2), jnp.int32),
            pltpu.SemaphoreType.DMA((2,)),
        ],
    )


def _l2rows(x):
    sq = jnp.sum(x * x, axis=1, keepdims=True)
    return x * lax.rsqrt(jnp.maximum(sq, 1e-12))


def _unpack_packed(a):
    hi = jnp.int32(-65536)
    e = lax.bitcast_convert_type(lax.shift_left(a, 16), jnp.float32)
    o = lax.bitcast_convert_type(a & hi, jnp.float32)
    return jnp.concatenate([e, o], axis=1)


def _finish_body(inv_s_ref, su_ref, sv_ref, wsv_ref, sqv_ref, wsu_ref, squ_ref,
                 wvagg_ref, wuagg_ref, wout_ref, out_ref):
    inv_s = inv_s_ref[0]
    u0 = _l2rows(_unpack_packed(su_ref[...]))
    i0 = _l2rows(_unpack_packed(sv_ref[...]))
    nv = wsv_ref[...] * lax.rsqrt(jnp.maximum(sqv_ref[...], 1e-12)) * inv_s
    nu = wsu_ref[...] * lax.rsqrt(jnp.maximum(squ_ref[...], 1e-12)) * inv_s
    hu = jnp.concatenate([u0, nv], axis=1)
    hi = jnp.concatenate([i0, nu], axis=1)
    uvec = _l2rows(jnp.maximum(
        jnp.dot(hu, wvagg_ref[...], preferred_element_type=jnp.float32), 0.0))
    ivec = _l2rows(jnp.maximum(
        jnp.dot(hi, wuagg_ref[...], preferred_element_type=jnp.float32), 0.0))
    out_ref[...] = jnp.dot(jnp.concatenate([uvec, ivec], axis=1),
                           wout_ref[...], preferred_element_type=jnp.float32)


def _finish(S, self_u, self_v, wsv, sqv, wsu, squ, Wv_agg, Wu_agg, Wout):
    B, D = self_u.shape
    inv_s = jnp.full((1,), 1.0 / S, jnp.float32)
    return pl.pallas_call(
        _finish_body,
        in_specs=[pl.BlockSpec(memory_space=pltpu.SMEM)] + [
            pl.BlockSpec(x.shape, lambda: (0,) * x.ndim)
            for x in (self_u, self_v, wsv, sqv, wsu, squ, Wv_agg, Wu_agg, Wout)],
        out_specs=pl.BlockSpec((B, Wout.shape[1]), lambda: (0, 0)),
        out_shape=jax.ShapeDtypeStruct((B, Wout.shape[1]), jnp.float32),
    )(inv_s, self_u, self_v, wsv, sqv, wsu, squ, Wv_agg, Wu_agg, Wout)


def kernel(u_features, v_features, Wu, Wv, Wout, i_edge_weights, u_edge_weights,
           Wv_agg, Wu_agg, u_indices, v_indices, u_supports, v_supports,
           user_support_val, item_support_val):
    B, S = u_supports.shape
    D = Wu.shape[0]
    NW = 32
    Bt = B // NW
    CB = 16

    half = D // 2
    perm = jnp.concatenate([
        jnp.arange(0, half, 2), jnp.arange(1, half, 2),
        jnp.arange(half, D, 2), jnp.arange(half + 1, D, 2)])

    sc_call = _make_sc_call(B, S, D, NW, Bt, CB)
    i32 = jnp.int32

    Tv = _transform_table(v_features, Wv, row_block=8192)
    self_v, wsv, sqv = sc_call(
        Tv, v_indices.astype(i32),
        v_supports.astype(i32).reshape(-1, 128),
        item_support_val.astype(i32).reshape(-1),
        i_edge_weights[:, perm].reshape(-1),
    )
    Tu = _transform_table(u_features, Wu, row_block=8192)
    self_u, wsu, squ = sc_call(
        Tu, u_indices.astype(i32),
        u_supports.astype(i32).reshape(-1, 128),
        user_support_val.astype(i32).reshape(-1),
        u_edge_weights[:, perm].reshape(-1),
    )

    perm2 = jnp.concatenate([jnp.arange(0, D, 2), jnp.arange(1, D, 2)])
    wvagg_p = jnp.concatenate([Wv_agg[:D][perm2], Wv_agg[D:][perm]])
    wuagg_p = jnp.concatenate([Wu_agg[:D][perm2], Wu_agg[D:][perm]])
    return _finish(S, self_u, self_v,
                   wsv.reshape(B, D), sqv.reshape(B, D),
                   wsu.reshape(B, D), squ.reshape(B, D),
                   wvagg_p, wuagg_p, Wout)


# speedup vs baseline: 1.2736x; 1.0092x over previous
import functools

import jax
import jax.numpy as jnp
from jax import lax
from jax.experimental import pallas as pl
from jax.experimental.pallas import tpu as pltpu
from jax.experimental.pallas import tpu_sc as plsc


def _transform_body(xt_ref, we_ref, wo_ref, t_ref):
    dn = (((0,), (0,)), ((), ()))
    hi = jnp.int32(-65536)
    x = xt_ref[...]
    e = jnp.maximum(lax.dot_general(
        x, we_ref[...], dn, preferred_element_type=jnp.float32), 0.0)
    o = jnp.maximum(lax.dot_general(
        x, wo_ref[...], dn, preferred_element_type=jnp.float32), 0.0)
    eb = lax.bitcast_convert_type(
        e.astype(jnp.bfloat16).astype(jnp.float32), jnp.int32)
    ob = lax.bitcast_convert_type(
        o.astype(jnp.bfloat16).astype(jnp.float32), jnp.int32)
    t_ref[...] = lax.shift_right_logical(eb, 16) | (ob & hi)


def _transform_table(features, W, row_block):
    n, d = features.shape
    h = d // 2
    grid = (n + row_block - 1) // row_block
    return pl.pallas_call(
        _transform_body,
        grid=(grid,),
        in_specs=[
            pl.BlockSpec((d, row_block), lambda i: (0, i)),
            pl.BlockSpec((d, h), lambda i: (0, 0)),
            pl.BlockSpec((d, h), lambda i: (0, 0)),
        ],
        out_specs=pl.BlockSpec((row_block, h), lambda i: (i, 0)),
        out_shape=jax.ShapeDtypeStruct((n, h), jnp.int32),
    )(features.T, W[:, 0::2], W[:, 1::2])


def _make_sc_call(B, S, D, NW, Bt, CB):
    NCH = Bt // CB
    G = (CB * S) // 128
    mesh = plsc.VectorSubcoreMesh(core_axis_name="c", subcore_axis_name="s")
    info = plsc.get_sparse_core_info()
    NC = info.num_cores

    def body(table, sidx, sup2, valf, wtab_hbm,
             self_out, ws_o, sq_o,
             sup_v, val_v, rows_v, wtab_v, ws_st, sq_st, sidx_v, srows_v,
             sems):
        wid = lax.axis_index("s") * NC + lax.axis_index("c")
        lane = jnp.arange(16, dtype=jnp.int32)

        def gather_self(table, idx_hbm, out_hbm):
            pltpu.sync_copy(idx_hbm.at[pl.ds(wid * Bt, Bt)], sidx_v)
            pltpu.async_copy(table.at[sidx_v], srows_v, sems.at[0]).wait()
            pltpu.sync_copy(srows_v, out_hbm.at[pl.ds(wid * Bt, Bt)])

        def do_side(table, sup2, valf, wtab_hbm, ws_out, sq_out):
            pltpu.sync_copy(wtab_hbm, wtab_v)

            def stage(ch, buf):
                row0 = wid * (Bt * S // 128) + ch * G
                pltpu.sync_copy(sup2.at[pl.ds(row0, G)], sup_v.at[buf])
                pltpu.sync_copy(
                    valf.at[pl.ds((wid * Bt + ch * CB) * S, CB * S)],
                    val_v.at[pl.ds(buf * CB * S, CB * S)])
                return [pltpu.async_copy(
                    table.at[sup_v.at[buf, g]],
                    rows_v.at[pl.ds((buf * G + g) * 128, 128)],
                    sems.at[buf]) for g in range(G)]

            def compute(ch, buf):
                hi = jnp.int32(-65536)
                dn = lax.GatherDimensionNumbers(
                    offset_dims=(), collapsed_slice_dims=(0,),
                    start_index_map=(0,))

                def b_body(b, _):
                    def sg_body(sg, carry):
                        base = (buf * CB + b) * S + sg * 16
                        wb = val_v[pl.ds(base, 16)] * D
                        accs = list(carry)
                        for j in range(16):
                            i = base + j
                            wbase = lax.gather(
                                wb, jnp.full((16, 1), j, jnp.int32), dn, (1,),
                                mode=lax.GatherScatterMode.PROMISE_IN_BOUNDS)
                            a01 = rows_v[i, pl.ds(0, 16)]
                            a23 = rows_v[i, pl.ds(16, 16)]
                            r0 = plsc.bitcast(a01 << 16, jnp.float32)
                            r1 = plsc.bitcast(a01 & hi, jnp.float32)
                            r2 = plsc.bitcast(a23 << 16, jnp.float32)
                            r3 = plsc.bitcast(a23 & hi, jnp.float32)
                            w0 = plsc.load_gather(wtab_v, [wbase + lane])
                            w1 = plsc.load_gather(wtab_v, [wbase + lane + 16])
                            w2 = plsc.load_gather(wtab_v, [wbase + lane + 32])
                            w3 = plsc.load_gather(wtab_v, [wbase + lane + 48])
                            accs = [accs[0] + r0 * w0, accs[1] + r1 * w1,
                                    accs[2] + r2 * w2, accs[3] + r3 * w3,
                                    accs[4] + r0 * r0, accs[5] + r1 * r1,
                                    accs[6] + r2 * r2, accs[7] + r3 * r3]
                        return tuple(accs)

                    z = jnp.zeros((16,), jnp.float32)
                    acc = lax.fori_loop(0, S // 16, sg_body, (z,) * 8)
                    row = ch * CB + b
                    for k in range(4):
                        ws_st[pl.ds(row * D + k * 16, 16)] = acc[k]
                        sq_st[pl.ds(row * D + k * 16, 16)] = acc[4 + k]
                    return 0

                lax.fori_loop(0, CB, b_body, 0)

            pending = stage(0, 0)
            for ch in range(NCH):
                nxt = None
                if ch + 1 < NCH:
                    nxt = stage(ch + 1, (ch + 1) % 2)
                for cp in pending:
                    cp.wait()
                compute(ch, ch % 2)
                pending = nxt
            pltpu.sync_copy(ws_st, ws_out.at[pl.ds(wid * Bt * D, Bt * D)])
            pltpu.sync_copy(sq_st, sq_out.at[pl.ds(wid * Bt * D, Bt * D)])

        gather_self(table, sidx, self_out)
        do_side(table, sup2, valf, wtab_hbm, ws_o, sq_o)

    return pl.kernel(
        body,
        out_type=[
            jax.ShapeDtypeStruct((B, D // 2), jnp.int32),
            jax.ShapeDtypeStruct((B * D,), jnp.float32),
            jax.ShapeDtypeStruct((B * D,), jnp.float32),
        ],
        mesh=mesh,
        compiler_params=pltpu.CompilerParams(
            use_tc_tiling_on_sc=False, needs_layout_passes=False),
        scratch_types=[
            pltpu.VMEM((2, G, 128), jnp.int32),
            pltpu.VMEM((2 * CB * S,), jnp.int32),
            pltpu.VMEM((2 * CB * S, D // 2), jnp.int32),
            pltpu.VMEM((5 * D,), jnp.float32),
            pltpu.VMEM((Bt * D,), jnp.float32),
            pltpu.VMEM((Bt * D,), jnp.float32),
            pltpu.VMEM((Bt,), jnp.int32),
            pltpu.VMEM((Bt, D // 2), jnp.int32),
            pltpu.SemaphoreType.DMA((2,)),
        ],
    )


def _l2rows(x):
    sq = jnp.sum(x * x, axis=1, keepdims=True)
    return x * lax.rsqrt(jnp.maximum(sq, 1e-12))


def _unpack_packed(a):
    hi = jnp.int32(-65536)
    e = lax.bitcast_convert_type(lax.shift_left(a, 16), jnp.float32)
    o = lax.bitcast_convert_type(a & hi, jnp.float32)
    return jnp.concatenate([e, o], axis=1)


def _finish_body(inv_s_ref, su_ref, sv_ref, wsv_ref, sqv_ref, wsu_ref, squ_ref,
                 wvagg_ref, wuagg_ref, wout_ref, out_ref):
    inv_s = inv_s_ref[0]
    u0 = _l2rows(_unpack_packed(su_ref[...]))
    i0 = _l2rows(_unpack_packed(sv_ref[...]))
    nv = wsv_ref[...] * lax.rsqrt(jnp.maximum(sqv_ref[...], 1e-12)) * inv_s
    nu = wsu_ref[...] * lax.rsqrt(jnp.maximum(squ_ref[...], 1e-12)) * inv_s
    hu = jnp.concatenate([u0, nv], axis=1)
    hi = jnp.concatenate([i0, nu], axis=1)
    uvec = _l2rows(jnp.maximum(
        jnp.dot(hu, wvagg_ref[...], preferred_element_type=jnp.float32), 0.0))
    ivec = _l2rows(jnp.maximum(
        jnp.dot(hi, wuagg_ref[...], preferred_element_type=jnp.float32), 0.0))
    out_ref[...] = jnp.dot(jnp.concatenate([uvec, ivec], axis=1),
                           wout_ref[...], preferred_element_type=jnp.float32)


def _finish(S, self_u, self_v, wsv, sqv, wsu, squ, Wv_agg, Wu_agg, Wout):
    B, D = self_u.shape
    inv_s = jnp.full((1,), 1.0 / S, jnp.float32)
    return pl.pallas_call(
        _finish_body,
        in_specs=[pl.BlockSpec(memory_space=pltpu.SMEM)] + [
            pl.BlockSpec(x.shape, lambda: (0,) * x.ndim)
            for x in (self_u, self_v, wsv, sqv, wsu, squ, Wv_agg, Wu_agg, Wout)],
        out_specs=pl.BlockSpec((B, Wout.shape[1]), lambda: (0, 0)),
        out_shape=jax.ShapeDtypeStruct((B, Wout.shape[1]), jnp.float32),
    )(inv_s, self_u, self_v, wsv, sqv, wsu, squ, Wv_agg, Wu_agg, Wout)


def kernel(u_features, v_features, Wu, Wv, Wout, i_edge_weights, u_edge_weights,
           Wv_agg, Wu_agg, u_indices, v_indices, u_supports, v_supports,
           user_support_val, item_support_val):
    B, S = u_supports.shape
    D = Wu.shape[0]
    NW = 32
    Bt = B // NW
    CB = 16

    half = D // 2
    perm = jnp.concatenate([
        jnp.arange(0, half, 2), jnp.arange(1, half, 2),
        jnp.arange(half, D, 2), jnp.arange(half + 1, D, 2)])

    sc_call = _make_sc_call(B, S, D, NW, Bt, CB)
    i32 = jnp.int32

    Tv = _transform_table(v_features, Wv, row_block=12800)
    self_v, wsv, sqv = sc_call(
        Tv, v_indices.astype(i32),
        v_supports.astype(i32).reshape(-1, 128),
        item_support_val.astype(i32).reshape(-1),
        i_edge_weights[:, perm].reshape(-1),
    )
    Tu = _transform_table(u_features, Wu, row_block=12800)
    self_u, wsu, squ = sc_call(
        Tu, u_indices.astype(i32),
        u_supports.astype(i32).reshape(-1, 128),
        user_support_val.astype(i32).reshape(-1),
        u_edge_weights[:, perm].reshape(-1),
    )

    perm2 = jnp.concatenate([jnp.arange(0, D, 2), jnp.arange(1, D, 2)])
    wvagg_p = jnp.concatenate([Wv_agg[:D][perm2], Wv_agg[D:][perm]])
    wuagg_p = jnp.concatenate([Wu_agg[:D][perm2], Wu_agg[D:][perm]])
    return _finish(S, self_u, self_v,
                   wsv.reshape(B, D), sqv.reshape(B, D),
                   wsu.reshape(B, D), squ.reshape(B, D),
                   wvagg_p, wuagg_p, Wout)


# speedup vs baseline: 1.2920x; 1.0145x over previous
import functools

import jax
import jax.numpy as jnp
from jax import lax
from jax.experimental import pallas as pl
from jax.experimental.pallas import tpu as pltpu
from jax.experimental.pallas import tpu_sc as plsc


def _transform_body(xt_ref, we_ref, wo_ref, t_ref):
    dn = (((0,), (0,)), ((), ()))
    hi = jnp.int32(-65536)
    x = xt_ref[...]
    e = jnp.maximum(lax.dot_general(
        x, we_ref[...], dn, preferred_element_type=jnp.float32), 0.0)
    o = jnp.maximum(lax.dot_general(
        x, wo_ref[...], dn, preferred_element_type=jnp.float32), 0.0)
    eb = lax.bitcast_convert_type(
        e.astype(jnp.bfloat16).astype(jnp.float32), jnp.int32)
    ob = lax.bitcast_convert_type(
        o.astype(jnp.bfloat16).astype(jnp.float32), jnp.int32)
    t_ref[...] = lax.shift_right_logical(eb, 16) | (ob & hi)


def _transform_table(features, W, row_block):
    n, d = features.shape
    h = d // 2
    grid = (n + row_block - 1) // row_block
    return pl.pallas_call(
        _transform_body,
        grid=(grid,),
        in_specs=[
            pl.BlockSpec((d, row_block), lambda i: (0, i)),
            pl.BlockSpec((d, h), lambda i: (0, 0)),
            pl.BlockSpec((d, h), lambda i: (0, 0)),
        ],
        out_specs=pl.BlockSpec((row_block, h), lambda i: (i, 0)),
        out_shape=jax.ShapeDtypeStruct((n, h), jnp.int32),
    )(features.T, W[:, 0::2], W[:, 1::2])


def _make_sc_call(B, S, D, NW, Bt, CB):
    NCH = Bt // CB
    G = (CB * S) // 128
    mesh = plsc.VectorSubcoreMesh(core_axis_name="c", subcore_axis_name="s")
    info = plsc.get_sparse_core_info()
    NC = info.num_cores

    def body(table, sidx, sup2, valf, wtab_hbm,
             self_out, ws_o, sq_o,
             sup_v, val_v, rows_v, wtab_v, ws_st, sq_st, sidx_v, srows_v,
             sems):
        wid = lax.axis_index("s") * NC + lax.axis_index("c")
        lane = jnp.arange(16, dtype=jnp.int32)

        def gather_self(table, idx_hbm, out_hbm):
            pltpu.sync_copy(idx_hbm.at[pl.ds(wid * Bt, Bt)], sidx_v)
            pltpu.async_copy(table.at[sidx_v], srows_v, sems.at[0]).wait()
            pltpu.sync_copy(srows_v, out_hbm.at[pl.ds(wid * Bt, Bt)])

        def do_side(table, sup2, valf, wtab_hbm, ws_out, sq_out):
            pltpu.sync_copy(wtab_hbm, wtab_v)

            def stage(ch, buf):
                row0 = wid * (Bt * S // 128) + ch * G
                pltpu.sync_copy(sup2.at[pl.ds(row0, G)], sup_v.at[buf])
                pltpu.sync_copy(
                    valf.at[pl.ds((wid * Bt + ch * CB) * S, CB * S)],
                    val_v.at[pl.ds(buf * CB * S, CB * S)])
                return [pltpu.async_copy(
                    table.at[sup_v.at[buf, g]],
                    rows_v.at[pl.ds((buf * G + g) * 128, 128)],
                    sems.at[buf]) for g in range(G)]

            def compute(ch, buf):
                hi = jnp.int32(-65536)
                dn = lax.GatherDimensionNumbers(
                    offset_dims=(), collapsed_slice_dims=(0,),
                    start_index_map=(0,))

                def b_body(b, _):
                    def sg_body(sg, carry):
                        base = (buf * CB + b) * S + sg * 16
                        wb = val_v[pl.ds(base, 16)] * D
                        accs = list(carry)
                        for j in range(16):
                            i = base + j
                            wbase = lax.gather(
                                wb, jnp.full((16, 1), j, jnp.int32), dn, (1,),
                                mode=lax.GatherScatterMode.PROMISE_IN_BOUNDS)
                            a01 = rows_v[i, pl.ds(0, 16)]
                            a23 = rows_v[i, pl.ds(16, 16)]
                            r0 = plsc.bitcast(a01 << 16, jnp.float32)
                            r1 = plsc.bitcast(a01 & hi, jnp.float32)
                            r2 = plsc.bitcast(a23 << 16, jnp.float32)
                            r3 = plsc.bitcast(a23 & hi, jnp.float32)
                            w0 = plsc.load_gather(wtab_v, [wbase + lane])
                            w1 = plsc.load_gather(wtab_v, [wbase + lane + 16])
                            w2 = plsc.load_gather(wtab_v, [wbase + lane + 32])
                            w3 = plsc.load_gather(wtab_v, [wbase + lane + 48])
                            accs = [accs[0] + r0 * w0, accs[1] + r1 * w1,
                                    accs[2] + r2 * w2, accs[3] + r3 * w3,
                                    accs[4] + r0 * r0, accs[5] + r1 * r1,
                                    accs[6] + r2 * r2, accs[7] + r3 * r3]
                        return tuple(accs)

                    z = jnp.zeros((16,), jnp.float32)
                    acc = lax.fori_loop(0, S // 16, sg_body, (z,) * 8)
                    row = ch * CB + b
                    for k in range(4):
                        ws_st[pl.ds(row * D + k * 16, 16)] = acc[k]
                        sq_st[pl.ds(row * D + k * 16, 16)] = acc[4 + k]
                    return 0

                lax.fori_loop(0, CB, b_body, 0)

            pending = stage(0, 0)
            for ch in range(NCH):
                nxt = None
                if ch + 1 < NCH:
                    nxt = stage(ch + 1, (ch + 1) % 2)
                for cp in pending:
                    cp.wait()
                compute(ch, ch % 2)
                pending = nxt
            pltpu.sync_copy(ws_st, ws_out.at[pl.ds(wid * Bt * D, Bt * D)])
            pltpu.sync_copy(sq_st, sq_out.at[pl.ds(wid * Bt * D, Bt * D)])

        gather_self(table, sidx, self_out)
        do_side(table, sup2, valf, wtab_hbm, ws_o, sq_o)

    return pl.kernel(
        body,
        out_type=[
            jax.ShapeDtypeStruct((B, D // 2), jnp.int32),
            jax.ShapeDtypeStruct((B * D,), jnp.float32),
            jax.ShapeDtypeStruct((B * D,), jnp.float32),
        ],
        mesh=mesh,
        compiler_params=pltpu.CompilerParams(
            use_tc_tiling_on_sc=False, needs_layout_passes=False),
        scratch_types=[
            pltpu.VMEM((2, G, 128), jnp.int32),
            pltpu.VMEM((2 * CB * S,), jnp.int32),
            pltpu.VMEM((2 * CB * S, D // 2), jnp.int32),
            pltpu.VMEM((5 * D,), jnp.float32),
            pltpu.VMEM((Bt * D,), jnp.float32),
            pltpu.VMEM((Bt * D,), jnp.float32),
            pltpu.VMEM((Bt,), jnp.int32),
            pltpu.VMEM((Bt, D // 2), jnp.int32),
            pltpu.SemaphoreType.DMA((2,)),
        ],
    )


def _l2rows(x):
    sq = jnp.sum(x * x, axis=1, keepdims=True)
    return x * lax.rsqrt(jnp.maximum(sq, 1e-12))


def _unpack_packed(a):
    hi = jnp.int32(-65536)
    e = lax.bitcast_convert_type(lax.shift_left(a, 16), jnp.float32)
    o = lax.bitcast_convert_type(a & hi, jnp.float32)
    return jnp.concatenate([e, o], axis=1)


def _finish_body(inv_s_ref, su_ref, sv_ref, wsv_ref, sqv_ref, wsu_ref, squ_ref,
                 wvagg_ref, wuagg_ref, wout_ref, out_ref):
    inv_s = inv_s_ref[0]
    u0 = _l2rows(_unpack_packed(su_ref[...]))
    i0 = _l2rows(_unpack_packed(sv_ref[...]))
    nv = wsv_ref[...] * lax.rsqrt(jnp.maximum(sqv_ref[...], 1e-12)) * inv_s
    nu = wsu_ref[...] * lax.rsqrt(jnp.maximum(squ_ref[...], 1e-12)) * inv_s
    hu = jnp.concatenate([u0, nv], axis=1)
    hi = jnp.concatenate([i0, nu], axis=1)
    uvec = _l2rows(jnp.maximum(
        jnp.dot(hu, wvagg_ref[...], preferred_element_type=jnp.float32), 0.0))
    ivec = _l2rows(jnp.maximum(
        jnp.dot(hi, wuagg_ref[...], preferred_element_type=jnp.float32), 0.0))
    out_ref[...] = jnp.dot(jnp.concatenate([uvec, ivec], axis=1),
                           wout_ref[...], preferred_element_type=jnp.float32)


def _finish(S, self_u, self_v, wsv, sqv, wsu, squ, Wv_agg, Wu_agg, Wout):
    B, D = self_u.shape
    inv_s = jnp.full((1,), 1.0 / S, jnp.float32)
    return pl.pallas_call(
        _finish_body,
        in_specs=[pl.BlockSpec(memory_space=pltpu.SMEM)] + [
            pl.BlockSpec(x.shape, lambda: (0,) * x.ndim)
            for x in (self_u, self_v, wsv, sqv, wsu, squ, Wv_agg, Wu_agg, Wout)],
        out_specs=pl.BlockSpec((B, Wout.shape[1]), lambda: (0, 0)),
        out_shape=jax.ShapeDtypeStruct((B, Wout.shape[1]), jnp.float32),
    )(inv_s, self_u, self_v, wsv, sqv, wsu, squ, Wv_agg, Wu_agg, Wout)


def kernel(u_features, v_features, Wu, Wv, Wout, i_edge_weights, u_edge_weights,
           Wv_agg, Wu_agg, u_indices, v_indices, u_supports, v_supports,
           user_support_val, item_support_val):
    B, S = u_supports.shape
    D = Wu.shape[0]
    NW = 32
    Bt = B // NW
    CB = 32

    half = D // 2
    perm = jnp.concatenate([
        jnp.arange(0, half, 2), jnp.arange(1, half, 2),
        jnp.arange(half, D, 2), jnp.arange(half + 1, D, 2)])

    sc_call = _make_sc_call(B, S, D, NW, Bt, CB)
    i32 = jnp.int32

    Tv = _transform_table(v_features, Wv, row_block=12800)
    self_v, wsv, sqv = sc_call(
        Tv, v_indices.astype(i32),
        v_supports.astype(i32).reshape(-1, 128),
        item_support_val.astype(i32).reshape(-1),
        i_edge_weights[:, perm].reshape(-1),
    )
    Tu = _transform_table(u_features, Wu, row_block=12800)
    self_u, wsu, squ = sc_call(
        Tu, u_indices.astype(i32),
        u_supports.astype(i32).reshape(-1, 128),
        user_support_val.astype(i32).reshape(-1),
        u_edge_weights[:, perm].reshape(-1),
    )

    perm2 = jnp.concatenate([jnp.arange(0, D, 2), jnp.arange(1, D, 2)])
    wvagg_p = jnp.concatenate([Wv_agg[:D][perm2], Wv_agg[D:][perm]])
    wuagg_p = jnp.concatenate([Wu_agg[:D][perm2], Wu_agg[D:][perm]])
    return _finish(S, self_u, self_v,
                   wsv.reshape(B, D), sqv.reshape(B, D),
                   wsu.reshape(B, D), squ.reshape(B, D),
                   wvagg_p, wuagg_p, Wout)


# speedup vs baseline: 1.3015x; 1.0073x over previous
import functools

import jax
import jax.numpy as jnp
from jax import lax
from jax.experimental import pallas as pl
from jax.experimental.pallas import tpu as pltpu
from jax.experimental.pallas import tpu_sc as plsc


def _transform_body(xt_ref, we_ref, wo_ref, t_ref):
    dn = (((0,), (0,)), ((), ()))
    hi = jnp.int32(-65536)
    x = xt_ref[...]
    e = jnp.maximum(lax.dot_general(
        x, we_ref[...], dn, preferred_element_type=jnp.float32), 0.0)
    o = jnp.maximum(lax.dot_general(
        x, wo_ref[...], dn, preferred_element_type=jnp.float32), 0.0)
    eb = lax.bitcast_convert_type(
        e.astype(jnp.bfloat16).astype(jnp.float32), jnp.int32)
    ob = lax.bitcast_convert_type(
        o.astype(jnp.bfloat16).astype(jnp.float32), jnp.int32)
    t_ref[...] = lax.shift_right_logical(eb, 16) | (ob & hi)


def _transform_table(features, W, row_block):
    n, d = features.shape
    h = d // 2
    grid = (n + row_block - 1) // row_block
    return pl.pallas_call(
        _transform_body,
        grid=(grid,),
        in_specs=[
            pl.BlockSpec((d, row_block), lambda i: (0, i)),
            pl.BlockSpec((d, h), lambda i: (0, 0)),
            pl.BlockSpec((d, h), lambda i: (0, 0)),
        ],
        out_specs=pl.BlockSpec((row_block, h), lambda i: (i, 0)),
        out_shape=jax.ShapeDtypeStruct((n, h), jnp.int32),
    )(features.T, W[:, 0::2], W[:, 1::2])


def _make_sc_call(B, S, D, NW, Bt, CB):
    NCH = Bt // CB
    G = (CB * S) // 128
    mesh = plsc.VectorSubcoreMesh(core_axis_name="c", subcore_axis_name="s")
    info = plsc.get_sparse_core_info()
    NC = info.num_cores

    def body(table, sidx, sup2, valf, wtab_hbm,
             self_out, ws_o, sq_o,
             sup_v, val_v, rows_v, wtab_v, ws_st, sq_st, sidx_v, srows_v,
             sems):
        wid = lax.axis_index("s") * NC + lax.axis_index("c")
        lane = jnp.arange(16, dtype=jnp.int32)

        def gather_self_start(table, idx_hbm):
            pltpu.sync_copy(idx_hbm.at[pl.ds(wid * Bt, Bt)], sidx_v)
            return pltpu.async_copy(table.at[sidx_v], srows_v, sems.at[2])

        def gather_self_finish(cp, out_hbm):
            cp.wait()
            pltpu.sync_copy(srows_v, out_hbm.at[pl.ds(wid * Bt, Bt)])

        def do_side(table, sup2, valf, wtab_hbm, ws_out, sq_out):
            pltpu.sync_copy(wtab_hbm, wtab_v)

            def stage(ch, buf):
                row0 = wid * (Bt * S // 128) + ch * G
                pltpu.sync_copy(sup2.at[pl.ds(row0, G)], sup_v.at[buf])
                pltpu.sync_copy(
                    valf.at[pl.ds((wid * Bt + ch * CB) * S, CB * S)],
                    val_v.at[pl.ds(buf * CB * S, CB * S)])
                return [pltpu.async_copy(
                    table.at[sup_v.at[buf, g]],
                    rows_v.at[pl.ds((buf * G + g) * 128, 128)],
                    sems.at[buf]) for g in range(G)]

            def compute(ch, buf):
                hi = jnp.int32(-65536)
                dn = lax.GatherDimensionNumbers(
                    offset_dims=(), collapsed_slice_dims=(0,),
                    start_index_map=(0,))

                def b_body(b, _):
                    def sg_body(sg, carry):
                        base = (buf * CB + b) * S + sg * 16
                        wb = val_v[pl.ds(base, 16)] * D
                        accs = list(carry)
                        for j in range(16):
                            i = base + j
                            wbase = lax.gather(
                                wb, jnp.full((16, 1), j, jnp.int32), dn, (1,),
                                mode=lax.GatherScatterMode.PROMISE_IN_BOUNDS)
                            a01 = rows_v[i, pl.ds(0, 16)]
                            a23 = rows_v[i, pl.ds(16, 16)]
                            r0 = plsc.bitcast(a01 << 16, jnp.float32)
                            r1 = plsc.bitcast(a01 & hi, jnp.float32)
                            r2 = plsc.bitcast(a23 << 16, jnp.float32)
                            r3 = plsc.bitcast(a23 & hi, jnp.float32)
                            w0 = plsc.load_gather(wtab_v, [wbase + lane])
                            w1 = plsc.load_gather(wtab_v, [wbase + lane + 16])
                            w2 = plsc.load_gather(wtab_v, [wbase + lane + 32])
                            w3 = plsc.load_gather(wtab_v, [wbase + lane + 48])
                            accs = [accs[0] + r0 * w0, accs[1] + r1 * w1,
                                    accs[2] + r2 * w2, accs[3] + r3 * w3,
                                    accs[4] + r0 * r0, accs[5] + r1 * r1,
                                    accs[6] + r2 * r2, accs[7] + r3 * r3]
                        return tuple(accs)

                    z = jnp.zeros((16,), jnp.float32)
                    acc = lax.fori_loop(0, S // 16, sg_body, (z,) * 8)
                    row = ch * CB + b
                    for k in range(4):
                        ws_st[pl.ds(row * D + k * 16, 16)] = acc[k]
                        sq_st[pl.ds(row * D + k * 16, 16)] = acc[4 + k]
                    return 0

                lax.fori_loop(0, CB, b_body, 0)

            pending = stage(0, 0)
            for ch in range(NCH):
                nxt = None
                if ch + 1 < NCH:
                    nxt = stage(ch + 1, (ch + 1) % 2)
                for cp in pending:
                    cp.wait()
                compute(ch, ch % 2)
                pending = nxt
            pltpu.sync_copy(ws_st, ws_out.at[pl.ds(wid * Bt * D, Bt * D)])
            pltpu.sync_copy(sq_st, sq_out.at[pl.ds(wid * Bt * D, Bt * D)])

        cp_self = gather_self_start(table, sidx)
        do_side(table, sup2, valf, wtab_hbm, ws_o, sq_o)
        gather_self_finish(cp_self, self_out)

    return pl.kernel(
        body,
        out_type=[
            jax.ShapeDtypeStruct((B, D // 2), jnp.int32),
            jax.ShapeDtypeStruct((B * D,), jnp.float32),
            jax.ShapeDtypeStruct((B * D,), jnp.float32),
        ],
        mesh=mesh,
        compiler_params=pltpu.CompilerParams(
            use_tc_tiling_on_sc=False, needs_layout_passes=False),
        scratch_types=[
            pltpu.VMEM((2, G, 128), jnp.int32),
            pltpu.VMEM((2 * CB * S,), jnp.int32),
            pltpu.VMEM((2 * CB * S, D // 2), jnp.int32),
            pltpu.VMEM((5 * D,), jnp.float32),
            pltpu.VMEM((Bt * D,), jnp.float32),
            pltpu.VMEM((Bt * D,), jnp.float32),
            pltpu.VMEM((Bt,), jnp.int32),
            pltpu.VMEM((Bt, D // 2), jnp.int32),
            pltpu.SemaphoreType.DMA((3,)),
        ],
    )


def _l2rows(x):
    sq = jnp.sum(x * x, axis=1, keepdims=True)
    return x * lax.rsqrt(jnp.maximum(sq, 1e-12))


def _unpack_packed(a):
    hi = jnp.int32(-65536)
    e = lax.bitcast_convert_type(lax.shift_left(a, 16), jnp.float32)
    o = lax.bitcast_convert_type(a & hi, jnp.float32)
    return jnp.concatenate([e, o], axis=1)


def _finish_body(inv_s_ref, su_ref, sv_ref, wsv_ref, sqv_ref, wsu_ref, squ_ref,
                 wvagg_ref, wuagg_ref, wout_ref, out_ref):
    inv_s = inv_s_ref[0]
    u0 = _l2rows(_unpack_packed(su_ref[...]))
    i0 = _l2rows(_unpack_packed(sv_ref[...]))
    nv = wsv_ref[...] * lax.rsqrt(jnp.maximum(sqv_ref[...], 1e-12)) * inv_s
    nu = wsu_ref[...] * lax.rsqrt(jnp.maximum(squ_ref[...], 1e-12)) * inv_s
    hu = jnp.concatenate([u0, nv], axis=1)
    hi = jnp.concatenate([i0, nu], axis=1)
    uvec = _l2rows(jnp.maximum(
        jnp.dot(hu, wvagg_ref[...], preferred_element_type=jnp.float32), 0.0))
    ivec = _l2rows(jnp.maximum(
        jnp.dot(hi, wuagg_ref[...], preferred_element_type=jnp.float32), 0.0))
    out_ref[...] = jnp.dot(jnp.concatenate([uvec, ivec], axis=1),
                           wout_ref[...], preferred_element_type=jnp.float32)


def _finish(S, self_u, self_v, wsv, sqv, wsu, squ, Wv_agg, Wu_agg, Wout):
    B, D = self_u.shape
    inv_s = jnp.full((1,), 1.0 / S, jnp.float32)
    return pl.pallas_call(
        _finish_body,
        in_specs=[pl.BlockSpec(memory_space=pltpu.SMEM)] + [
            pl.BlockSpec(x.shape, lambda: (0,) * x.ndim)
            for x in (self_u, self_v, wsv, sqv, wsu, squ, Wv_agg, Wu_agg, Wout)],
        out_specs=pl.BlockSpec((B, Wout.shape[1]), lambda: (0, 0)),
        out_shape=jax.ShapeDtypeStruct((B, Wout.shape[1]), jnp.float32),
    )(inv_s, self_u, self_v, wsv, sqv, wsu, squ, Wv_agg, Wu_agg, Wout)


def kernel(u_features, v_features, Wu, Wv, Wout, i_edge_weights, u_edge_weights,
           Wv_agg, Wu_agg, u_indices, v_indices, u_supports, v_supports,
           user_support_val, item_support_val):
    B, S = u_supports.shape
    D = Wu.shape[0]
    NW = 32
    Bt = B // NW
    CB = 32

    half = D // 2
    perm = jnp.concatenate([
        jnp.arange(0, half, 2), jnp.arange(1, half, 2),
        jnp.arange(half, D, 2), jnp.arange(half + 1, D, 2)])

    sc_call = _make_sc_call(B, S, D, NW, Bt, CB)
    i32 = jnp.int32

    Tv = _transform_table(v_features, Wv, row_block=12800)
    self_v, wsv, sqv = sc_call(
        Tv, v_indices.astype(i32),
        v_supports.astype(i32).reshape(-1, 128),
        item_support_val.astype(i32).reshape(-1),
        i_edge_weights[:, perm].reshape(-1),
    )
    Tu = _transform_table(u_features, Wu, row_block=12800)
    self_u, wsu, squ = sc_call(
        Tu, u_indices.astype(i32),
        u_supports.astype(i32).reshape(-1, 128),
        user_support_val.astype(i32).reshape(-1),
        u_edge_weights[:, perm].reshape(-1),
    )

    perm2 = jnp.concatenate([jnp.arange(0, D, 2), jnp.arange(1, D, 2)])
    wvagg_p = jnp.concatenate([Wv_agg[:D][perm2], Wv_agg[D:][perm]])
    wuagg_p = jnp.concatenate([Wu_agg[:D][perm2], Wu_agg[D:][perm]])
    return _finish(S, self_u, self_v,
                   wsv.reshape(B, D), sqv.reshape(B, D),
                   wsu.reshape(B, D), squ.reshape(B, D),
                   wvagg_p, wuagg_p, Wout)


# speedup vs baseline: 1.3105x; 1.0070x over previous
import functools

import jax
import jax.numpy as jnp
from jax import lax
from jax.experimental import pallas as pl
from jax.experimental.pallas import tpu as pltpu
from jax.experimental.pallas import tpu_sc as plsc


def _transform_body(xt_ref, we_ref, wo_ref, t_ref):
    dn = (((0,), (0,)), ((), ()))
    hi = jnp.int32(-65536)
    x = xt_ref[...]
    e = jnp.maximum(lax.dot_general(
        x, we_ref[...], dn, preferred_element_type=jnp.float32), 0.0)
    o = jnp.maximum(lax.dot_general(
        x, wo_ref[...], dn, preferred_element_type=jnp.float32), 0.0)
    eb = lax.bitcast_convert_type(
        e.astype(jnp.bfloat16).astype(jnp.float32), jnp.int32)
    ob = lax.bitcast_convert_type(
        o.astype(jnp.bfloat16).astype(jnp.float32), jnp.int32)
    t_ref[...] = lax.shift_right_logical(eb, 16) | (ob & hi)


def _transform_table(features, W, row_block):
    n, d = features.shape
    h = d // 2
    grid = (n + row_block - 1) // row_block
    return pl.pallas_call(
        _transform_body,
        grid=(grid,),
        in_specs=[
            pl.BlockSpec((d, row_block), lambda i: (0, i)),
            pl.BlockSpec((d, h), lambda i: (0, 0)),
            pl.BlockSpec((d, h), lambda i: (0, 0)),
        ],
        out_specs=pl.BlockSpec((row_block, h), lambda i: (i, 0)),
        out_shape=jax.ShapeDtypeStruct((n, h), jnp.int32),
    )(features.T, W[:, 0::2], W[:, 1::2])


def _make_sc_call(B, S, D, NW, Bt, CB):
    NCH = Bt // CB
    G = (CB * S) // 128
    mesh = plsc.VectorSubcoreMesh(core_axis_name="c", subcore_axis_name="s")
    info = plsc.get_sparse_core_info()
    NC = info.num_cores

    def body(table, sidx, sup2, valf, wtab_hbm,
             self_out, ws_o, sq_o,
             sup_v, val_v, rows_v, wtab_v, ws_st, sq_st, sidx_v, srows_v,
             sems):
        wid = lax.axis_index("s") * NC + lax.axis_index("c")
        lane = jnp.arange(16, dtype=jnp.int32)

        def gather_self_start(table, idx_hbm):
            pltpu.sync_copy(idx_hbm.at[pl.ds(wid * Bt, Bt)], sidx_v)
            return pltpu.async_copy(table.at[sidx_v], srows_v, sems.at[2])

        def gather_self_finish(cp, out_hbm):
            cp.wait()
            pltpu.sync_copy(srows_v, out_hbm.at[pl.ds(wid * Bt, Bt)])

        def do_side(table, sup2, valf, wtab_hbm, ws_out, sq_out):
            pltpu.sync_copy(wtab_hbm, wtab_v)

            def stage(ch, buf):
                row0 = wid * (Bt * S // 128) + ch * G
                pltpu.sync_copy(sup2.at[pl.ds(row0, G)], sup_v.at[buf])
                pltpu.sync_copy(
                    valf.at[pl.ds((wid * Bt + ch * CB) * S, CB * S)],
                    val_v.at[pl.ds(buf * CB * S, CB * S)])
                return [pltpu.async_copy(
                    table.at[sup_v.at[buf, g]],
                    rows_v.at[pl.ds((buf * G + g) * 128, 128)],
                    sems.at[buf]) for g in range(G)]

            def compute(ch, buf):
                hi = jnp.int32(-65536)
                dn = lax.GatherDimensionNumbers(
                    offset_dims=(), collapsed_slice_dims=(0,),
                    start_index_map=(0,))

                def b_body(b, _):
                    def sg_body(sg, carry):
                        base = (buf * CB + b) * S + sg * 16
                        wb = val_v[pl.ds(base, 16)] * D
                        accs = list(carry)
                        for j in range(16):
                            i = base + j
                            wbase = lax.gather(
                                wb, jnp.full((16, 1), j, jnp.int32), dn, (1,),
                                mode=lax.GatherScatterMode.PROMISE_IN_BOUNDS)
                            a01 = rows_v[i, pl.ds(0, 16)]
                            a23 = rows_v[i, pl.ds(16, 16)]
                            r0 = plsc.bitcast(a01 << 16, jnp.float32)
                            r1 = plsc.bitcast(a01 & hi, jnp.float32)
                            r2 = plsc.bitcast(a23 << 16, jnp.float32)
                            r3 = plsc.bitcast(a23 & hi, jnp.float32)
                            w0 = plsc.load_gather(wtab_v, [wbase + lane])
                            w1 = plsc.load_gather(wtab_v, [wbase + lane + 16])
                            w2 = plsc.load_gather(wtab_v, [wbase + lane + 32])
                            w3 = plsc.load_gather(wtab_v, [wbase + lane + 48])
                            accs = [accs[0] + r0 * w0, accs[1] + r1 * w1,
                                    accs[2] + r2 * w2, accs[3] + r3 * w3,
                                    accs[4] + r0 * r0, accs[5] + r1 * r1,
                                    accs[6] + r2 * r2, accs[7] + r3 * r3]
                        return tuple(accs)

                    z = jnp.zeros((16,), jnp.float32)
                    acc = lax.fori_loop(0, S // 16, sg_body, (z,) * 8)
                    row = ch * CB + b
                    for k in range(4):
                        ws_st[pl.ds(row * D + k * 16, 16)] = acc[k]
                        sq_st[pl.ds(row * D + k * 16, 16)] = acc[4 + k]
                    return 0

                lax.fori_loop(0, CB, b_body, 0)

            pending = stage(0, 0)
            for ch in range(NCH):
                nxt = None
                if ch + 1 < NCH:
                    nxt = stage(ch + 1, (ch + 1) % 2)
                for cp in pending:
                    cp.wait()
                compute(ch, ch % 2)
                pending = nxt
            pltpu.sync_copy(ws_st, ws_out.at[pl.ds(wid * Bt * D, Bt * D)])
            pltpu.sync_copy(sq_st, sq_out.at[pl.ds(wid * Bt * D, Bt * D)])

        cp_self = gather_self_start(table, sidx)
        do_side(table, sup2, valf, wtab_hbm, ws_o, sq_o)
        gather_self_finish(cp_self, self_out)

    return pl.kernel(
        body,
        out_type=[
            jax.ShapeDtypeStruct((B, D // 2), jnp.int32),
            jax.ShapeDtypeStruct((B * D,), jnp.float32),
            jax.ShapeDtypeStruct((B * D,), jnp.float32),
        ],
        mesh=mesh,
        compiler_params=pltpu.CompilerParams(
            use_tc_tiling_on_sc=False, needs_layout_passes=False),
        scratch_types=[
            pltpu.VMEM((2, G, 128), jnp.int32),
            pltpu.VMEM((2 * CB * S,), jnp.int32),
            pltpu.VMEM((2 * CB * S, D // 2), jnp.int32),
            pltpu.VMEM((5 * D,), jnp.float32),
            pltpu.VMEM((Bt * D,), jnp.float32),
            pltpu.VMEM((Bt * D,), jnp.float32),
            pltpu.VMEM((Bt,), jnp.int32),
            pltpu.VMEM((Bt, D // 2), jnp.int32),
            pltpu.SemaphoreType.DMA((3,)),
        ],
    )


def _l2rows(x):
    sq = jnp.sum(x * x, axis=1, keepdims=True)
    return x * lax.rsqrt(jnp.maximum(sq, 1e-12))


def _unpack_packed(a):
    hi = jnp.int32(-65536)
    e = lax.bitcast_convert_type(lax.shift_left(a, 16), jnp.float32)
    o = lax.bitcast_convert_type(a & hi, jnp.float32)
    return jnp.concatenate([e, o], axis=1)


def _finish_body(inv_s_ref, su_ref, sv_ref, wsv_ref, sqv_ref, wsu_ref, squ_ref,
                 wvagg_ref, wuagg_ref, wout_ref, out_ref):
    inv_s = inv_s_ref[0]
    u0 = _l2rows(_unpack_packed(su_ref[...]))
    i0 = _l2rows(_unpack_packed(sv_ref[...]))
    nv = wsv_ref[...] * lax.rsqrt(jnp.maximum(sqv_ref[...], 1e-12)) * inv_s
    nu = wsu_ref[...] * lax.rsqrt(jnp.maximum(squ_ref[...], 1e-12)) * inv_s
    hu = jnp.concatenate([u0, nv], axis=1)
    hi = jnp.concatenate([i0, nu], axis=1)
    uvec = _l2rows(jnp.maximum(
        jnp.dot(hu, wvagg_ref[...], preferred_element_type=jnp.float32), 0.0))
    ivec = _l2rows(jnp.maximum(
        jnp.dot(hi, wuagg_ref[...], preferred_element_type=jnp.float32), 0.0))
    out_ref[...] = jnp.dot(jnp.concatenate([uvec, ivec], axis=1),
                           wout_ref[...], preferred_element_type=jnp.float32)


def _finish(S, self_u, self_v, wsv, sqv, wsu, squ, Wv_agg, Wu_agg, Wout):
    B, D = self_u.shape
    inv_s = jnp.full((1,), 1.0 / S, jnp.float32)
    return pl.pallas_call(
        _finish_body,
        in_specs=[pl.BlockSpec(memory_space=pltpu.SMEM)] + [
            pl.BlockSpec(x.shape, lambda: (0,) * x.ndim)
            for x in (self_u, self_v, wsv, sqv, wsu, squ, Wv_agg, Wu_agg, Wout)],
        out_specs=pl.BlockSpec((B, Wout.shape[1]), lambda: (0, 0)),
        out_shape=jax.ShapeDtypeStruct((B, Wout.shape[1]), jnp.float32),
    )(inv_s, self_u, self_v, wsv, sqv, wsu, squ, Wv_agg, Wu_agg, Wout)


def kernel(u_features, v_features, Wu, Wv, Wout, i_edge_weights, u_edge_weights,
           Wv_agg, Wu_agg, u_indices, v_indices, u_supports, v_supports,
           user_support_val, item_support_val):
    B, S = u_supports.shape
    D = Wu.shape[0]
    NW = 32
    Bt = B // NW
    CB = 32

    half = D // 2
    perm = jnp.concatenate([
        jnp.arange(0, half, 2), jnp.arange(1, half, 2),
        jnp.arange(half, D, 2), jnp.arange(half + 1, D, 2)])

    sc_call = _make_sc_call(B, S, D, NW, Bt, CB)
    i32 = jnp.int32

    Tv = _transform_table(v_features, Wv, row_block=25600)
    self_v, wsv, sqv = sc_call(
        Tv, v_indices.astype(i32),
        v_supports.astype(i32).reshape(-1, 128),
        item_support_val.astype(i32).reshape(-1),
        i_edge_weights[:, perm].reshape(-1),
    )
    Tu = _transform_table(u_features, Wu, row_block=25600)
    self_u, wsu, squ = sc_call(
        Tu, u_indices.astype(i32),
        u_supports.astype(i32).reshape(-1, 128),
        user_support_val.astype(i32).reshape(-1),
        u_edge_weights[:, perm].reshape(-1),
    )

    perm2 = jnp.concatenate([jnp.arange(0, D, 2), jnp.arange(1, D, 2)])
    wvagg_p = jnp.concatenate([Wv_agg[:D][perm2], Wv_agg[D:][perm]])
    wuagg_p = jnp.concatenate([Wu_agg[:D][perm2], Wu_agg[D:][perm]])
    return _finish(S, self_u, self_v,
                   wsv.reshape(B, D), sqv.reshape(B, D),
                   wsu.reshape(B, D), squ.reshape(B, D),
                   wvagg_p, wuagg_p, Wout)


# speedup vs baseline: 1.3108x; 1.0002x over previous
import jax
import jax.numpy as jnp
from jax import lax
from jax.experimental import pallas as pl
from jax.experimental.pallas import tpu as pltpu
from jax.experimental.pallas import tpu_sc as plsc


def _transform_body(xt_ref, we_ref, wo_ref, t_ref):
    dn = (((0,), (0,)), ((), ()))
    hi = jnp.int32(-65536)
    x = xt_ref[...]
    e = jnp.maximum(lax.dot_general(
        x, we_ref[...], dn, preferred_element_type=jnp.float32), 0.0)
    o = jnp.maximum(lax.dot_general(
        x, wo_ref[...], dn, preferred_element_type=jnp.float32), 0.0)
    eb = lax.bitcast_convert_type(
        e.astype(jnp.bfloat16).astype(jnp.float32), jnp.int32)
    ob = lax.bitcast_convert_type(
        o.astype(jnp.bfloat16).astype(jnp.float32), jnp.int32)
    t_ref[...] = lax.shift_right_logical(eb, 16) | (ob & hi)


def _transform_table(features, W, row_block):
    n, d = features.shape
    h = d // 2
    grid = (n + row_block - 1) // row_block
    return pl.pallas_call(
        _transform_body,
        grid=(grid,),
        in_specs=[
            pl.BlockSpec((d, row_block), lambda i: (0, i)),
            pl.BlockSpec((d, h), lambda i: (0, 0)),
            pl.BlockSpec((d, h), lambda i: (0, 0)),
        ],
        out_specs=pl.BlockSpec((row_block, h), lambda i: (i, 0)),
        out_shape=jax.ShapeDtypeStruct((n, h), jnp.int32),
    )(features.T, W[:, 0::2], W[:, 1::2])


def _make_sc_call(B, S, D, NW, Bt, CB):
    NCH = Bt // CB
    G = (CB * S) // 128
    mesh = plsc.VectorSubcoreMesh(core_axis_name="c", subcore_axis_name="s")
    info = plsc.get_sparse_core_info()
    NC = info.num_cores

    def body(table, sidx, sup2, valf, wtab_hbm,
             self_out, ws_o, sq_o,
             sup_v, val_v, rows_v, wtab_v, ws_st, sq_st, sidx_v, srows_v,
             sems):
        wid = lax.axis_index("s") * NC + lax.axis_index("c")
        lane = jnp.arange(16, dtype=jnp.int32)

        def gather_self_start(table, idx_hbm):
            pltpu.sync_copy(idx_hbm.at[pl.ds(wid * Bt, Bt)], sidx_v)
            return pltpu.async_copy(table.at[sidx_v], srows_v, sems.at[2])

        def gather_self_finish(cp, out_hbm):
            cp.wait()
            pltpu.sync_copy(srows_v, out_hbm.at[pl.ds(wid * Bt, Bt)])

        def do_side(table, sup2, valf, wtab_hbm, ws_out, sq_out):
            pltpu.sync_copy(wtab_hbm, wtab_v)

            def stage(ch, buf):
                row0 = wid * (Bt * S // 128) + ch * G
                pltpu.sync_copy(sup2.at[pl.ds(row0, G)], sup_v.at[buf])
                pltpu.sync_copy(
                    valf.at[pl.ds((wid * Bt + ch * CB) * S, CB * S)],
                    val_v.at[pl.ds(buf * CB * S, CB * S)])
                return [pltpu.async_copy(
                    table.at[sup_v.at[buf, g]],
                    rows_v.at[pl.ds((buf * G + g) * 128, 128)],
                    sems.at[buf]) for g in range(G)]

            def compute(ch, buf):
                hi = jnp.int32(-65536)
                dn = lax.GatherDimensionNumbers(
                    offset_dims=(), collapsed_slice_dims=(0,),
                    start_index_map=(0,))

                def b_body(b, _):
                    def sg_body(sg, carry):
                        base = (buf * CB + b) * S + sg * 16
                        wb = val_v[pl.ds(base, 16)] * D
                        accs = list(carry)
                        for j in range(16):
                            i = base + j
                            wbase = lax.gather(
                                wb, jnp.full((16, 1), j, jnp.int32), dn, (1,),
                                mode=lax.GatherScatterMode.PROMISE_IN_BOUNDS)
                            a01 = rows_v[i, pl.ds(0, 16)]
                            a23 = rows_v[i, pl.ds(16, 16)]
                            r0 = plsc.bitcast(a01 << 16, jnp.float32)
                            r1 = plsc.bitcast(a01 & hi, jnp.float32)
                            r2 = plsc.bitcast(a23 << 16, jnp.float32)
                            r3 = plsc.bitcast(a23 & hi, jnp.float32)
                            w0 = plsc.load_gather(wtab_v, [wbase + lane])
                            w1 = plsc.load_gather(wtab_v, [wbase + lane + 16])
                            w2 = plsc.load_gather(wtab_v, [wbase + lane + 32])
                            w3 = plsc.load_gather(wtab_v, [wbase + lane + 48])
                            accs = [accs[0] + r0 * w0, accs[1] + r1 * w1,
                                    accs[2] + r2 * w2, accs[3] + r3 * w3,
                                    accs[4] + r0 * r0, accs[5] + r1 * r1,
                                    accs[6] + r2 * r2, accs[7] + r3 * r3]
                        return tuple(accs)

                    z = jnp.zeros((16,), jnp.float32)
                    acc = lax.fori_loop(0, S // 16, sg_body, (z,) * 8)
                    row = ch * CB + b
                    for k in range(4):
                        ws_st[pl.ds(row * D + k * 16, 16)] = acc[k]
                        sq_st[pl.ds(row * D + k * 16, 16)] = acc[4 + k]
                    return 0

                lax.fori_loop(0, CB, b_body, 0)

            pending = stage(0, 0)
            for ch in range(NCH):
                nxt = None
                if ch + 1 < NCH:
                    nxt = stage(ch + 1, (ch + 1) % 2)
                for cp in pending:
                    cp.wait()
                compute(ch, ch % 2)
                pending = nxt
            pltpu.sync_copy(ws_st, ws_out.at[pl.ds(wid * Bt * D, Bt * D)])
            pltpu.sync_copy(sq_st, sq_out.at[pl.ds(wid * Bt * D, Bt * D)])

        cp_self = gather_self_start(table, sidx)
        do_side(table, sup2, valf, wtab_hbm, ws_o, sq_o)
        gather_self_finish(cp_self, self_out)

    return pl.kernel(
        body,
        out_type=[
            jax.ShapeDtypeStruct((B, D // 2), jnp.int32),
            jax.ShapeDtypeStruct((B * D,), jnp.float32),
            jax.ShapeDtypeStruct((B * D,), jnp.float32),
        ],
        mesh=mesh,
        compiler_params=pltpu.CompilerParams(
            use_tc_tiling_on_sc=False, needs_layout_passes=False),
        scratch_types=[
            pltpu.VMEM((2, G, 128), jnp.int32),
            pltpu.VMEM((2 * CB * S,), jnp.int32),
            pltpu.VMEM((2 * CB * S, D // 2), jnp.int32),
            pltpu.VMEM((5 * D,), jnp.float32),
            pltpu.VMEM((Bt * D,), jnp.float32),
            pltpu.VMEM((Bt * D,), jnp.float32),
            pltpu.VMEM((Bt,), jnp.int32),
            pltpu.VMEM((Bt, D // 2), jnp.int32),
            pltpu.SemaphoreType.DMA((3,)),
        ],
    )


def _l2rows(x):
    sq = jnp.sum(x * x, axis=1, keepdims=True)
    return x * lax.rsqrt(jnp.maximum(sq, 1e-12))


def _unpack_packed(a):
    hi = jnp.int32(-65536)
    e = lax.bitcast_convert_type(lax.shift_left(a, 16), jnp.float32)
    o = lax.bitcast_convert_type(a & hi, jnp.float32)
    return jnp.concatenate([e, o], axis=1)


def _finish_body(inv_s_ref, su_ref, sv_ref, wsv_ref, sqv_ref, wsu_ref, squ_ref,
                 wvagg_ref, wuagg_ref, wout_ref, out_ref):
    inv_s = inv_s_ref[0]
    u0 = _l2rows(_unpack_packed(su_ref[...]))
    i0 = _l2rows(_unpack_packed(sv_ref[...]))
    nv = wsv_ref[...] * lax.rsqrt(jnp.maximum(sqv_ref[...], 1e-12)) * inv_s
    nu = wsu_ref[...] * lax.rsqrt(jnp.maximum(squ_ref[...], 1e-12)) * inv_s
    hu = jnp.concatenate([u0, nv], axis=1)
    hi = jnp.concatenate([i0, nu], axis=1)
    uvec = _l2rows(jnp.maximum(
        jnp.dot(hu, wvagg_ref[...], preferred_element_type=jnp.float32), 0.0))
    ivec = _l2rows(jnp.maximum(
        jnp.dot(hi, wuagg_ref[...], preferred_element_type=jnp.float32), 0.0))
    out_ref[...] = jnp.dot(jnp.concatenate([uvec, ivec], axis=1),
                           wout_ref[...], preferred_element_type=jnp.float32)


def _finish(S, self_u, self_v, wsv, sqv, wsu, squ, Wv_agg, Wu_agg, Wout):
    B, D = self_u.shape
    inv_s = jnp.full((1,), 1.0 / S, jnp.float32)
    return pl.pallas_call(
        _finish_body,
        in_specs=[pl.BlockSpec(memory_space=pltpu.SMEM)] + [
            pl.BlockSpec(x.shape, lambda: (0,) * x.ndim)
            for x in (self_u, self_v, wsv, sqv, wsu, squ, Wv_agg, Wu_agg, Wout)],
        out_specs=pl.BlockSpec((B, Wout.shape[1]), lambda: (0, 0)),
        out_shape=jax.ShapeDtypeStruct((B, Wout.shape[1]), jnp.float32),
    )(inv_s, self_u, self_v, wsv, sqv, wsu, squ, Wv_agg, Wu_agg, Wout)


def kernel(u_features, v_features, Wu, Wv, Wout, i_edge_weights, u_edge_weights,
           Wv_agg, Wu_agg, u_indices, v_indices, u_supports, v_supports,
           user_support_val, item_support_val):
    B, S = u_supports.shape
    D = Wu.shape[0]
    NW = 32
    Bt = B // NW
    CB = 32

    half = D // 2
    perm = jnp.concatenate([
        jnp.arange(0, half, 2), jnp.arange(1, half, 2),
        jnp.arange(half, D, 2), jnp.arange(half + 1, D, 2)])

    sc_call = _make_sc_call(B, S, D, NW, Bt, CB)
    i32 = jnp.int32

    Tv = _transform_table(v_features, Wv, row_block=25600)
    self_v, wsv, sqv = sc_call(
        Tv, v_indices.astype(i32),
        v_supports.astype(i32).reshape(-1, 128),
        item_support_val.astype(i32).reshape(-1),
        i_edge_weights[:, perm].reshape(-1),
    )
    Tu = _transform_table(u_features, Wu, row_block=25600)
    self_u, wsu, squ = sc_call(
        Tu, u_indices.astype(i32),
        u_supports.astype(i32).reshape(-1, 128),
        user_support_val.astype(i32).reshape(-1),
        u_edge_weights[:, perm].reshape(-1),
    )

    perm2 = jnp.concatenate([jnp.arange(0, D, 2), jnp.arange(1, D, 2)])
    wvagg_p = jnp.concatenate([Wv_agg[:D][perm2], Wv_agg[D:][perm]])
    wuagg_p = jnp.concatenate([Wu_agg[:D][perm2], Wu_agg[D:][perm]])
    return _finish(S, self_u, self_v,
                   wsv.reshape(B, D), sqv.reshape(B, D),
                   wsu.reshape(B, D), squ.reshape(B, D),
                   wvagg_p, wuagg_p, Wout)
